# baseline (speedup 1.0000x reference)
import functools
import jax
import jax.numpy as jnp
from jax import lax
from jax.experimental import pallas as pl
from jax.experimental.pallas import tpu as pltpu
from jax.experimental.pallas import tpu_sc as plsc

N_NODES = 10000
E = 160000
D = 128
D_VAL = 16
NUM_FES = 16
H1 = 512
H_FC = 64
CHUNK = 128
NCHUNKS = E // CHUNK
NC, NS = 2, 16
NW = NC * NS
ITERS = (NCHUNKS + NW - 1) // NW
NR_CHUNK = 80
NRCHUNKS = N_NODES // NR_CHUNK
NR_ITERS = (NRCHUNKS + NS - 1) // NS

_mesh = plsc.VectorSubcoreMesh(core_axis_name="c", subcore_axis_name="s")


def _gather_body(hn_hbm, src_hbm, dst_hbm, hns_hbm, hnd_hbm,
                 idx_s, idx_d, rows_s, rows_d, sem):
    cid = lax.axis_index("c")
    sid = lax.axis_index("s")
    wid = sid * NC + cid

    def body(i, carry):
        c = wid + i * NW

        @pl.when(c < NCHUNKS)
        def _():
            base = c * CHUNK
            pltpu.sync_copy(src_hbm.at[pl.ds(base, CHUNK)], idx_s)
            pltpu.sync_copy(dst_hbm.at[pl.ds(base, CHUNK)], idx_d)
            ca = pltpu.async_copy(hn_hbm.at[idx_s], rows_s, sem)
            cb = pltpu.async_copy(hn_hbm.at[idx_d], rows_d, sem)
            ca.wait()
            cb.wait()
            pltpu.sync_copy(rows_s, hns_hbm.at[pl.ds(base, CHUNK)])
            pltpu.sync_copy(rows_d, hnd_hbm.at[pl.ds(base, CHUNK)])

        return carry

    lax.fori_loop(0, ITERS, body, 0)


_gather = pl.kernel(
    _gather_body,
    mesh=_mesh,
    out_type=[jax.ShapeDtypeStruct((E, D), jnp.float32),
              jax.ShapeDtypeStruct((E, D), jnp.float32)],
    scratch_types=[
        pltpu.VMEM((CHUNK,), jnp.int32),
        pltpu.VMEM((CHUNK,), jnp.int32),
        pltpu.VMEM((CHUNK, D), jnp.float32),
        pltpu.VMEM((CHUNK, D), jnp.float32),
        pltpu.SemaphoreType.DMA,
    ],
)


def _scatter_body(henw_hbm, dst_hbm, zeros_hbm, out_hbm, idx2, rows, acc):
    cid = lax.axis_index("c")
    sid = lax.axis_index("s")
    wid = sid * NC + cid

    def zbody(i, carry):
        c = sid + i * NS

        @pl.when(c < NRCHUNKS)
        def _():
            pltpu.sync_copy(zeros_hbm, acc.at[pl.ds(c * NR_CHUNK, NR_CHUNK)])

        return carry

    lax.fori_loop(0, NR_ITERS, zbody, 0)
    plsc.subcore_barrier()

    def body(i, carry):
        c = wid + i * NW

        @pl.when(c < NCHUNKS)
        def _():
            base = c * CHUNK
            pltpu.sync_copy(dst_hbm.at[pl.ds(base, CHUNK)], idx2.at[0])
            pltpu.sync_copy(henw_hbm.at[pl.ds(base, CHUNK)], rows)
            pltpu.sync_copy(rows, acc.at[idx2.at[0]], add=True)

        return carry

    lax.fori_loop(0, ITERS, body, 0)
    plsc.subcore_barrier()

    def dbody(i, carry):
        c = sid + i * NS

        @pl.when(c < NRCHUNKS)
        def _():
            pltpu.sync_copy(acc.at[pl.ds(c * NR_CHUNK, NR_CHUNK)],
                            out_hbm.at[cid, pl.ds(c * NR_CHUNK, NR_CHUNK)])

        return carry

    lax.fori_loop(0, NR_ITERS, dbody, 0)


_scatter = pl.kernel(
    _scatter_body,
    mesh=_mesh,
    out_type=jax.ShapeDtypeStruct((NC, N_NODES, D), jnp.float32),
    scratch_types=[
        pltpu.VMEM((1, CHUNK), jnp.int32),
        pltpu.VMEM((CHUNK, D), jnp.float32),
        pltpu.VMEM_SHARED((N_NODES, D), jnp.float32),
    ],
)


BE = 640


def _edge_body(he_r, hns_r, hnd_r, fes_r, fn_r,
               w1a_r, w1b_r, w1c_r, b1_r, w2_r, b2_r, fw1_r, fw2_r,
               hen_r, henw_r):
    bf = jnp.bfloat16
    t = jnp.dot(he_r[:].astype(bf), w1a_r[:], preferred_element_type=jnp.float32)
    t = t + jnp.dot(hns_r[:].astype(bf), w1b_r[:], preferred_element_type=jnp.float32)
    t = t + jnp.dot(hnd_r[:].astype(bf), w1c_r[:], preferred_element_type=jnp.float32)
    t = jnp.maximum(t + b1_r[:], 0.0).astype(bf)
    v = jnp.dot(t, w2_r[:], preferred_element_type=jnp.float32) + b2_r[:]
    h = jnp.maximum(
        jnp.dot(fes_r[:].astype(bf), fw1_r[:],
                preferred_element_type=jnp.float32) * 0.25,
        0.0).astype(bf)
    a = jnp.dot(h, fw2_r[:], preferred_element_type=jnp.float32)
    heu = v[:, 0:1] * a[:, 0:D]
    for i in range(1, D_VAL):
        heu = heu + v[:, i:i + 1] * a[:, i * D:(i + 1) * D]
    hen = he_r[:] + heu * (fn_r[:, 0:1] * (1.0 / 32.0))
    hen_r[:] = hen
    henw_r[:] = hen * fn_r[:, 1:2]


def _edge_call(he, hns, hnd, fes, fn, w1a, w1b, w1c, b1, w2, b2, fw1, fw2):
    blk = lambda r, c: pl.BlockSpec((r, c), lambda i: (i, 0))
    full = lambda r, c: pl.BlockSpec((r, c), lambda i: (0, 0))
    return pl.pallas_call(
        _edge_body,
        grid=(E // BE,),
        in_specs=[
            blk(BE, D), blk(BE, D), blk(BE, D), blk(BE, NUM_FES), blk(BE, 2),
            full(D, H1), full(D, H1), full(D, H1), full(1, H1),
            full(H1, D_VAL), full(1, D_VAL),
            full(NUM_FES, H_FC), full(H_FC, D_VAL * D),
        ],
        out_specs=[blk(BE, D), blk(BE, D)],
        out_shape=[jax.ShapeDtypeStruct((E, D), jnp.float32),
                   jax.ShapeDtypeStruct((E, D), jnp.float32)],
    )(he, hns, hnd, fes, fn, w1a, w1b, w1c, b1, w2, b2, fw1, fw2)


BN = 1000


def _node_body(hn_r, pr_r, w1a_r, w1b_r, b1_r, w2_r, b2_r, hnn_r):
    nt = pr_r[0] + pr_r[1]
    u = jnp.dot(hn_r[:], w1a_r[:], preferred_element_type=jnp.float32)
    u = u + jnp.dot(nt, w1b_r[:], preferred_element_type=jnp.float32)
    u = jnp.maximum(u + b1_r[:], 0.0)
    hnn_r[:] = hn_r[:] + jnp.dot(u, w2_r[:],
                                 preferred_element_type=jnp.float32) + b2_r[:]


def _node_call(hn, partials, w1a, w1b, b1, w2, b2):
    return pl.pallas_call(
        _node_body,
        grid=(N_NODES // BN,),
        in_specs=[
            pl.BlockSpec((BN, D), lambda i: (i, 0)),
            pl.BlockSpec((NC, BN, D), lambda i: (0, i, 0)),
            pl.BlockSpec((D, H1), lambda i: (0, 0)),
            pl.BlockSpec((D, H1), lambda i: (0, 0)),
            pl.BlockSpec((1, H1), lambda i: (0, 0)),
            pl.BlockSpec((H1, D), lambda i: (0, 0)),
            pl.BlockSpec((1, D), lambda i: (0, 0)),
        ],
        out_specs=pl.BlockSpec((BN, D), lambda i: (i, 0)),
        out_shape=jax.ShapeDtypeStruct((N_NODES, D), jnp.float32),
    )(hn, partials, w1a, w1b, b1, w2, b2)


@jax.jit
def kernel(hn, he, edge_index, fe, fes, norm,
           ev_W1, ev_b1, ev_W2, ev_b2, fc_W1, fc_W2,
           nu_W1, nu_b1, nu_W2, nu_b2):
    src = edge_index[0]
    dst = edge_index[1]
    hns, hnd = _gather(hn, src, dst)
    fn = jnp.concatenate([fe, norm[:, None]], axis=1)
    bf = jnp.bfloat16
    hen, henw = _edge_call(
        he, hns, hnd, fes, fn,
        ev_W1[:D].astype(bf), ev_W1[D:2 * D].astype(bf),
        ev_W1[2 * D:].astype(bf),
        ev_b1.reshape(1, H1), ev_W2.astype(bf), ev_b2.reshape(1, D_VAL),
        fc_W1.astype(bf), fc_W2.astype(bf))
    partials = _scatter(henw, dst, jnp.zeros((NR_CHUNK, D), jnp.float32))
    hnn = _node_call(hn, partials,
                     nu_W1[:D], nu_W1[D:], nu_b1.reshape(1, H1),
                     nu_W2, nu_b2.reshape(1, D))
    return hnn, hen


# speedup vs baseline: 1.2372x; 1.2372x over previous
import functools
import jax
import jax.numpy as jnp
from jax import lax
from jax.experimental import pallas as pl
from jax.experimental.pallas import tpu as pltpu
from jax.experimental.pallas import tpu_sc as plsc

N_NODES = 10000
E = 160000
D = 128
D_VAL = 16
NUM_FES = 16
H1 = 512
H_FC = 64
CHUNK = 128
NCHUNKS = E // CHUNK
NC, NS = 2, 16
NW = NC * NS
ITERS = (NCHUNKS + NW - 1) // NW
NR_CHUNK = 80
NRCHUNKS = N_NODES // NR_CHUNK
NR_ITERS = (NRCHUNKS + NS - 1) // NS

_mesh = plsc.VectorSubcoreMesh(core_axis_name="c", subcore_axis_name="s")


def _gather_body(hn_hbm, src_hbm, dst_hbm, hns_hbm, hnd_hbm,
                 idx_s, idx_d, rows_s, rows_d, sem):
    cid = lax.axis_index("c")
    sid = lax.axis_index("s")
    wid = sid * NC + cid

    def body(i, carry):
        c = wid + i * NW

        @pl.when(c < NCHUNKS)
        def _():
            base = c * CHUNK
            pltpu.sync_copy(src_hbm.at[pl.ds(base, CHUNK)], idx_s)
            pltpu.sync_copy(dst_hbm.at[pl.ds(base, CHUNK)], idx_d)
            ca = pltpu.async_copy(hn_hbm.at[idx_s], rows_s, sem)
            cb = pltpu.async_copy(hn_hbm.at[idx_d], rows_d, sem)
            ca.wait()
            cb.wait()
            pltpu.sync_copy(rows_s, hns_hbm.at[pl.ds(base, CHUNK)])
            pltpu.sync_copy(rows_d, hnd_hbm.at[pl.ds(base, CHUNK)])

        return carry

    lax.fori_loop(0, ITERS, body, 0)


_gather = pl.kernel(
    _gather_body,
    mesh=_mesh,
    out_type=[jax.ShapeDtypeStruct((E, D), jnp.float32),
              jax.ShapeDtypeStruct((E, D), jnp.float32)],
    scratch_types=[
        pltpu.VMEM((CHUNK,), jnp.int32),
        pltpu.VMEM((CHUNK,), jnp.int32),
        pltpu.VMEM((CHUNK, D), jnp.float32),
        pltpu.VMEM((CHUNK, D), jnp.float32),
        pltpu.SemaphoreType.DMA,
    ],
)


def _scatter_body(henw_hbm, dst_hbm, zeros_hbm, out_hbm, idx2, rows, acc):
    cid = lax.axis_index("c")
    sid = lax.axis_index("s")
    wid = sid * NC + cid

    def zbody(i, carry):
        c = sid + i * NS

        @pl.when(c < NRCHUNKS)
        def _():
            pltpu.sync_copy(zeros_hbm, acc.at[pl.ds(c * NR_CHUNK, NR_CHUNK)])

        return carry

    lax.fori_loop(0, NR_ITERS, zbody, 0)
    plsc.subcore_barrier()

    def body(i, carry):
        c = wid + i * NW

        @pl.when(c < NCHUNKS)
        def _():
            base = c * CHUNK
            pltpu.sync_copy(dst_hbm.at[pl.ds(base, CHUNK)], idx2.at[0])
            pltpu.sync_copy(henw_hbm.at[pl.ds(base, CHUNK)], rows)
            pltpu.sync_copy(rows, acc.at[idx2.at[0]], add=True)

        return carry

    lax.fori_loop(0, ITERS, body, 0)
    plsc.subcore_barrier()

    def dbody(i, carry):
        c = sid + i * NS

        @pl.when(c < NRCHUNKS)
        def _():
            pltpu.sync_copy(acc.at[pl.ds(c * NR_CHUNK, NR_CHUNK)],
                            out_hbm.at[cid, pl.ds(c * NR_CHUNK, NR_CHUNK)])

        return carry

    lax.fori_loop(0, NR_ITERS, dbody, 0)


_scatter = pl.kernel(
    _scatter_body,
    mesh=_mesh,
    out_type=jax.ShapeDtypeStruct((NC, N_NODES, D), jnp.float32),
    scratch_types=[
        pltpu.VMEM((1, CHUNK), jnp.int32),
        pltpu.VMEM((CHUNK, D), jnp.float32),
        pltpu.VMEM_SHARED((N_NODES, D), jnp.float32),
    ],
)


BE = 640


def _edge_body(he_r, hns_r, hnd_r, fes_r, fn_r,
               w1a_r, w1b_r, w1c_r, b1_r, w2_r, b2_r, fw1_r, fw2_r, s_r,
               hen_r, henw_r):
    bf = jnp.bfloat16
    t = jnp.dot(he_r[:].astype(bf), w1a_r[:], preferred_element_type=jnp.float32)
    t = t + jnp.dot(hns_r[:].astype(bf), w1b_r[:], preferred_element_type=jnp.float32)
    t = t + jnp.dot(hnd_r[:].astype(bf), w1c_r[:], preferred_element_type=jnp.float32)
    t = jnp.maximum(t + b1_r[:], 0.0).astype(bf)
    v = jnp.dot(t, w2_r[:], preferred_element_type=jnp.float32) + b2_r[:]
    h = jnp.maximum(
        jnp.dot(fes_r[:].astype(bf), fw1_r[:],
                preferred_element_type=jnp.float32) * 0.25,
        0.0).astype(bf)
    a = jnp.dot(h, fw2_r[:], preferred_element_type=jnp.float32)
    v_s = (v * (fn_r[:, 0:1] * (1.0 / 32.0))).astype(bf)
    vb = jnp.dot(v_s, s_r[:], preferred_element_type=jnp.float32)
    prod = vb * a
    heu = prod[:, 0:D]
    for i in range(1, D_VAL):
        heu = heu + prod[:, i * D:(i + 1) * D]
    hen = he_r[:] + heu
    hen_r[:] = hen
    henw_r[:] = hen * fn_r[:, 1:2]


def _edge_call(he, hns, hnd, fes, fn, w1a, w1b, w1c, b1, w2, b2, fw1, fw2, s):
    blk = lambda r, c: pl.BlockSpec((r, c), lambda i: (i, 0))
    full = lambda r, c: pl.BlockSpec((r, c), lambda i: (0, 0))
    return pl.pallas_call(
        _edge_body,
        grid=(E // BE,),
        in_specs=[
            blk(BE, D), blk(BE, D), blk(BE, D), blk(BE, NUM_FES), blk(BE, 2),
            full(D, H1), full(D, H1), full(D, H1), full(1, H1),
            full(H1, D_VAL), full(1, D_VAL),
            full(NUM_FES, H_FC), full(H_FC, D_VAL * D), full(D_VAL, D_VAL * D),
        ],
        out_specs=[blk(BE, D), blk(BE, D)],
        out_shape=[jax.ShapeDtypeStruct((E, D), jnp.float32),
                   jax.ShapeDtypeStruct((E, D), jnp.float32)],
    )(he, hns, hnd, fes, fn, w1a, w1b, w1c, b1, w2, b2, fw1, fw2, s)


BN = 1000


def _node_body(hn_r, pr_r, w1a_r, w1b_r, b1_r, w2_r, b2_r, hnn_r):
    nt = pr_r[0] + pr_r[1]
    u = jnp.dot(hn_r[:], w1a_r[:], preferred_element_type=jnp.float32)
    u = u + jnp.dot(nt, w1b_r[:], preferred_element_type=jnp.float32)
    u = jnp.maximum(u + b1_r[:], 0.0)
    hnn_r[:] = hn_r[:] + jnp.dot(u, w2_r[:],
                                 preferred_element_type=jnp.float32) + b2_r[:]


def _node_call(hn, partials, w1a, w1b, b1, w2, b2):
    return pl.pallas_call(
        _node_body,
        grid=(N_NODES // BN,),
        in_specs=[
            pl.BlockSpec((BN, D), lambda i: (i, 0)),
            pl.BlockSpec((NC, BN, D), lambda i: (0, i, 0)),
            pl.BlockSpec((D, H1), lambda i: (0, 0)),
            pl.BlockSpec((D, H1), lambda i: (0, 0)),
            pl.BlockSpec((1, H1), lambda i: (0, 0)),
            pl.BlockSpec((H1, D), lambda i: (0, 0)),
            pl.BlockSpec((1, D), lambda i: (0, 0)),
        ],
        out_specs=pl.BlockSpec((BN, D), lambda i: (i, 0)),
        out_shape=jax.ShapeDtypeStruct((N_NODES, D), jnp.float32),
    )(hn, partials, w1a, w1b, b1, w2, b2)


@jax.jit
def kernel(hn, he, edge_index, fe, fes, norm,
           ev_W1, ev_b1, ev_W2, ev_b2, fc_W1, fc_W2,
           nu_W1, nu_b1, nu_W2, nu_b2):
    src = edge_index[0]
    dst = edge_index[1]
    hns, hnd = _gather(hn, src, dst)
    fn = jnp.concatenate([fe, norm[:, None]], axis=1)
    bf = jnp.bfloat16
    hen, henw = _edge_call(
        he, hns, hnd, fes, fn,
        ev_W1[:D].astype(bf), ev_W1[D:2 * D].astype(bf),
        ev_W1[2 * D:].astype(bf),
        ev_b1.reshape(1, H1), ev_W2.astype(bf), ev_b2.reshape(1, D_VAL),
        fc_W1.astype(bf), fc_W2.astype(bf),
        jnp.kron(jnp.eye(D_VAL, dtype=jnp.float32),
                 jnp.ones((1, D), jnp.float32)).astype(bf))
    partials = _scatter(henw, dst, jnp.zeros((NR_CHUNK, D), jnp.float32))
    hnn = _node_call(hn, partials,
                     nu_W1[:D], nu_W1[D:], nu_b1.reshape(1, H1),
                     nu_W2, nu_b2.reshape(1, D))
    return hnn, hen


# speedup vs baseline: 1.2595x; 1.0181x over previous
import functools
import jax
import jax.numpy as jnp
from jax import lax
from jax.experimental import pallas as pl
from jax.experimental.pallas import tpu as pltpu
from jax.experimental.pallas import tpu_sc as plsc

N_NODES = 10000
E = 160000
D = 128
D_VAL = 16
NUM_FES = 16
H1 = 512
H_FC = 64
CHUNK = 128
NCHUNKS = E // CHUNK
NC, NS = 2, 16
NW = NC * NS
ITERS = (NCHUNKS + NW - 1) // NW
NR_CHUNK = 80
NRCHUNKS = N_NODES // NR_CHUNK
NR_ITERS = (NRCHUNKS + NS - 1) // NS

_mesh = plsc.VectorSubcoreMesh(core_axis_name="c", subcore_axis_name="s")


def _gather_body(hn_hbm, src_hbm, dst_hbm, hns_hbm, hnd_hbm,
                 idx_s, idx_d, rows_s, rows_d, sem):
    cid = lax.axis_index("c")
    sid = lax.axis_index("s")
    wid = sid * NC + cid

    def body(i, carry):
        c = wid + i * NW

        @pl.when(c < NCHUNKS)
        def _():
            base = c * CHUNK
            pltpu.sync_copy(src_hbm.at[pl.ds(base, CHUNK)], idx_s)
            pltpu.sync_copy(dst_hbm.at[pl.ds(base, CHUNK)], idx_d)
            ca = pltpu.async_copy(hn_hbm.at[idx_s], rows_s, sem)
            cb = pltpu.async_copy(hn_hbm.at[idx_d], rows_d, sem)
            ca.wait()
            cb.wait()
            pltpu.sync_copy(rows_s, hns_hbm.at[pl.ds(base, CHUNK)])
            pltpu.sync_copy(rows_d, hnd_hbm.at[pl.ds(base, CHUNK)])

        return carry

    lax.fori_loop(0, ITERS, body, 0)


_gather = pl.kernel(
    _gather_body,
    mesh=_mesh,
    out_type=[jax.ShapeDtypeStruct((E, D), jnp.float32),
              jax.ShapeDtypeStruct((E, D), jnp.float32)],
    scratch_types=[
        pltpu.VMEM((CHUNK,), jnp.int32),
        pltpu.VMEM((CHUNK,), jnp.int32),
        pltpu.VMEM((CHUNK, D), jnp.float32),
        pltpu.VMEM((CHUNK, D), jnp.float32),
        pltpu.SemaphoreType.DMA,
    ],
)


def _scatter_body(henw_hbm, dst_hbm, zeros_hbm, out_hbm, idx2, rows, acc):
    cid = lax.axis_index("c")
    sid = lax.axis_index("s")
    wid = sid * NC + cid

    def zbody(i, carry):
        c = sid + i * NS

        @pl.when(c < NRCHUNKS)
        def _():
            pltpu.sync_copy(zeros_hbm, acc.at[pl.ds(c * NR_CHUNK, NR_CHUNK)])

        return carry

    lax.fori_loop(0, NR_ITERS, zbody, 0)
    plsc.subcore_barrier()

    def body(i, carry):
        c = wid + i * NW

        @pl.when(c < NCHUNKS)
        def _():
            base = c * CHUNK
            pltpu.sync_copy(dst_hbm.at[pl.ds(base, CHUNK)], idx2.at[0])
            pltpu.sync_copy(henw_hbm.at[pl.ds(base, CHUNK)], rows)
            pltpu.sync_copy(rows, acc.at[idx2.at[0]], add=True)

        return carry

    lax.fori_loop(0, ITERS, body, 0)
    plsc.subcore_barrier()

    def dbody(i, carry):
        c = sid + i * NS

        @pl.when(c < NRCHUNKS)
        def _():
            pltpu.sync_copy(acc.at[pl.ds(c * NR_CHUNK, NR_CHUNK)],
                            out_hbm.at[cid, pl.ds(c * NR_CHUNK, NR_CHUNK)])

        return carry

    lax.fori_loop(0, NR_ITERS, dbody, 0)


_scatter = pl.kernel(
    _scatter_body,
    mesh=_mesh,
    out_type=jax.ShapeDtypeStruct((NC, N_NODES, D), jnp.float32),
    scratch_types=[
        pltpu.VMEM((1, CHUNK), jnp.int32),
        pltpu.VMEM((CHUNK, D), jnp.float32),
        pltpu.VMEM_SHARED((N_NODES, D), jnp.float32),
    ],
)


BE = 640


def _edge_body(he_r, hns_r, hnd_r, fes_r, fn_r,
               w1_r, b1_r, w2_r, b2_r, fw1_r, fw2_r, s_r,
               hen_r, henw_r):
    bf = jnp.bfloat16
    z = jnp.concatenate(
        [he_r[:].astype(bf), hns_r[:].astype(bf), hnd_r[:].astype(bf)], axis=1)
    t = jnp.dot(z, w1_r[:], preferred_element_type=jnp.float32)
    t = jnp.maximum(t + b1_r[:], 0.0).astype(bf)
    v = jnp.dot(t, w2_r[:], preferred_element_type=jnp.float32) + b2_r[:]
    h = jnp.maximum(
        jnp.dot(fes_r[:].astype(bf), fw1_r[:],
                preferred_element_type=jnp.float32) * 0.25,
        0.0).astype(bf)
    a = jnp.dot(h, fw2_r[:], preferred_element_type=jnp.float32)
    v_s = (v * (fn_r[:, 0:1] * (1.0 / 32.0))).astype(bf)
    vb = jnp.dot(v_s, s_r[:], preferred_element_type=jnp.float32)
    prod = vb * a
    heu = prod[:, 0:D]
    for i in range(1, D_VAL):
        heu = heu + prod[:, i * D:(i + 1) * D]
    hen = he_r[:] + heu
    hen_r[:] = hen
    henw_r[:] = hen * fn_r[:, 1:2]


def _edge_call(he, hns, hnd, fes, fn, w1, b1, w2, b2, fw1, fw2, s):
    blk = lambda r, c: pl.BlockSpec((r, c), lambda i: (i, 0))
    full = lambda r, c: pl.BlockSpec((r, c), lambda i: (0, 0))
    return pl.pallas_call(
        _edge_body,
        grid=(E // BE,),
        in_specs=[
            blk(BE, D), blk(BE, D), blk(BE, D), blk(BE, NUM_FES), blk(BE, 2),
            full(3 * D, H1), full(1, H1),
            full(H1, D_VAL), full(1, D_VAL),
            full(NUM_FES, H_FC), full(H_FC, D_VAL * D), full(D_VAL, D_VAL * D),
        ],
        out_specs=[blk(BE, D), blk(BE, D)],
        out_shape=[jax.ShapeDtypeStruct((E, D), jnp.float32),
                   jax.ShapeDtypeStruct((E, D), jnp.float32)],
    )(he, hns, hnd, fes, fn, w1, b1, w2, b2, fw1, fw2, s)


BN = 1000


def _node_body(hn_r, pr_r, w1a_r, w1b_r, b1_r, w2_r, b2_r, hnn_r):
    nt = pr_r[0] + pr_r[1]
    u = jnp.dot(hn_r[:], w1a_r[:], preferred_element_type=jnp.float32)
    u = u + jnp.dot(nt, w1b_r[:], preferred_element_type=jnp.float32)
    u = jnp.maximum(u + b1_r[:], 0.0)
    hnn_r[:] = hn_r[:] + jnp.dot(u, w2_r[:],
                                 preferred_element_type=jnp.float32) + b2_r[:]


def _node_call(hn, partials, w1a, w1b, b1, w2, b2):
    return pl.pallas_call(
        _node_body,
        grid=(N_NODES // BN,),
        in_specs=[
            pl.BlockSpec((BN, D), lambda i: (i, 0)),
            pl.BlockSpec((NC, BN, D), lambda i: (0, i, 0)),
            pl.BlockSpec((D, H1), lambda i: (0, 0)),
            pl.BlockSpec((D, H1), lambda i: (0, 0)),
            pl.BlockSpec((1, H1), lambda i: (0, 0)),
            pl.BlockSpec((H1, D), lambda i: (0, 0)),
            pl.BlockSpec((1, D), lambda i: (0, 0)),
        ],
        out_specs=pl.BlockSpec((BN, D), lambda i: (i, 0)),
        out_shape=jax.ShapeDtypeStruct((N_NODES, D), jnp.float32),
    )(hn, partials, w1a, w1b, b1, w2, b2)


@jax.jit
def kernel(hn, he, edge_index, fe, fes, norm,
           ev_W1, ev_b1, ev_W2, ev_b2, fc_W1, fc_W2,
           nu_W1, nu_b1, nu_W2, nu_b2):
    src = edge_index[0]
    dst = edge_index[1]
    hns, hnd = _gather(hn, src, dst)
    fn = jnp.concatenate([fe, norm[:, None]], axis=1)
    bf = jnp.bfloat16
    hen, henw = _edge_call(
        he, hns, hnd, fes, fn,
        ev_W1.astype(bf),
        ev_b1.reshape(1, H1), ev_W2.astype(bf), ev_b2.reshape(1, D_VAL),
        fc_W1.astype(bf), fc_W2.astype(bf),
        jnp.kron(jnp.eye(D_VAL, dtype=jnp.float32),
                 jnp.ones((1, D), jnp.float32)).astype(bf))
    partials = _scatter(henw, dst, jnp.zeros((NR_CHUNK, D), jnp.float32))
    hnn = _node_call(hn, partials,
                     nu_W1[:D], nu_W1[D:], nu_b1.reshape(1, H1),
                     nu_W2, nu_b2.reshape(1, D))
    return hnn, hen


# speedup vs baseline: 1.3662x; 1.0847x over previous
import functools
import jax
import jax.numpy as jnp
from jax import lax
from jax.experimental import pallas as pl
from jax.experimental.pallas import tpu as pltpu
from jax.experimental.pallas import tpu_sc as plsc

N_NODES = 10000
E = 160000
D = 128
D_VAL = 16
NUM_FES = 16
H1 = 512
H_FC = 64
CHUNK = 128
NCHUNKS = E // CHUNK
NC, NS = 2, 16
NW = NC * NS
ITERS = (NCHUNKS + NW - 1) // NW
NR_CHUNK = 80
NRCHUNKS = N_NODES // NR_CHUNK
NR_ITERS = (NRCHUNKS + NS - 1) // NS

_mesh = plsc.VectorSubcoreMesh(core_axis_name="c", subcore_axis_name="s")


def _gather_body(hn_hbm, src_hbm, dst_hbm, hns_hbm, hnd_hbm,
                 idx_s, idx_d, rows_s, rows_d, sem):
    cid = lax.axis_index("c")
    sid = lax.axis_index("s")
    wid = sid * NC + cid

    def body(i, carry):
        c = wid + i * NW

        @pl.when(c < NCHUNKS)
        def _():
            base = c * CHUNK
            pltpu.sync_copy(src_hbm.at[pl.ds(base, CHUNK)], idx_s)
            pltpu.sync_copy(dst_hbm.at[pl.ds(base, CHUNK)], idx_d)
            ca = pltpu.async_copy(hn_hbm.at[idx_s], rows_s, sem)
            cb = pltpu.async_copy(hn_hbm.at[idx_d], rows_d, sem)
            ca.wait()
            cb.wait()
            pltpu.sync_copy(rows_s, hns_hbm.at[pl.ds(base, CHUNK)])
            pltpu.sync_copy(rows_d, hnd_hbm.at[pl.ds(base, CHUNK)])

        return carry

    lax.fori_loop(0, ITERS, body, 0)


_gather = pl.kernel(
    _gather_body,
    mesh=_mesh,
    out_type=[jax.ShapeDtypeStruct((E, D), jnp.float32),
              jax.ShapeDtypeStruct((E, D), jnp.float32)],
    scratch_types=[
        pltpu.VMEM((CHUNK,), jnp.int32),
        pltpu.VMEM((CHUNK,), jnp.int32),
        pltpu.VMEM((CHUNK, D), jnp.float32),
        pltpu.VMEM((CHUNK, D), jnp.float32),
        pltpu.SemaphoreType.DMA,
    ],
)


def _scatter_body(henw_hbm, dst_hbm, zeros_hbm, out_hbm, idx2, rows, acc):
    cid = lax.axis_index("c")
    sid = lax.axis_index("s")
    wid = sid * NC + cid

    def zbody(i, carry):
        c = sid + i * NS

        @pl.when(c < NRCHUNKS)
        def _():
            pltpu.sync_copy(zeros_hbm, acc.at[pl.ds(c * NR_CHUNK, NR_CHUNK)])

        return carry

    lax.fori_loop(0, NR_ITERS, zbody, 0)
    plsc.subcore_barrier()

    def body(i, carry):
        c = wid + i * NW

        @pl.when(c < NCHUNKS)
        def _():
            base = c * CHUNK
            pltpu.sync_copy(dst_hbm.at[pl.ds(base, CHUNK)], idx2.at[0])
            pltpu.sync_copy(henw_hbm.at[pl.ds(base, CHUNK)], rows)
            pltpu.sync_copy(rows, acc.at[idx2.at[0]], add=True)

        return carry

    lax.fori_loop(0, ITERS, body, 0)
    plsc.subcore_barrier()

    def dbody(i, carry):
        c = sid + i * NS

        @pl.when(c < NRCHUNKS)
        def _():
            pltpu.sync_copy(acc.at[pl.ds(c * NR_CHUNK, NR_CHUNK)],
                            out_hbm.at[cid, pl.ds(c * NR_CHUNK, NR_CHUNK)])

        return carry

    lax.fori_loop(0, NR_ITERS, dbody, 0)


_scatter = pl.kernel(
    _scatter_body,
    mesh=_mesh,
    out_type=jax.ShapeDtypeStruct((NC, N_NODES, D), jnp.float32),
    scratch_types=[
        pltpu.VMEM((1, CHUNK), jnp.int32),
        pltpu.VMEM((CHUNK, D), jnp.float32),
        pltpu.VMEM_SHARED((N_NODES, D), jnp.float32),
    ],
)


BE = 640


def _edge_body(he_r, hns_r, hnd_r, fes_r, fn_r,
               w1_r, b1_r, w2_r, b2_r, fw1_r, r_r, s_r, w2r_r,
               hen_r, henw_r):
    bf = jnp.bfloat16
    z = jnp.concatenate(
        [he_r[:].astype(bf), hns_r[:].astype(bf), hnd_r[:].astype(bf)], axis=1)
    t = jnp.dot(z, w1_r[:], preferred_element_type=jnp.float32)
    t = jnp.maximum(t + b1_r[:], 0.0).astype(bf)
    v = jnp.dot(t, w2_r[:], preferred_element_type=jnp.float32) + b2_r[:]
    h = jnp.maximum(
        jnp.dot(fes_r[:].astype(bf), fw1_r[:],
                preferred_element_type=jnp.float32) * 0.25,
        0.0).astype(bf)
    v_s = (v * (fn_r[:, 0:1] * (1.0 / 32.0))).astype(bf)
    h_rep = jnp.dot(h, r_r[:], preferred_element_type=jnp.float32)
    v_tile = jnp.dot(v_s, s_r[:], preferred_element_type=jnp.float32)
    p = (h_rep * v_tile).astype(bf)
    heu = jnp.dot(p, w2r_r[:], preferred_element_type=jnp.float32)
    hen = he_r[:] + heu
    hen_r[:] = hen
    henw_r[:] = hen * fn_r[:, 1:2]


def _edge_call(he, hns, hnd, fes, fn, w1, b1, w2, b2, fw1, r, s, w2r):
    blk = lambda rr, c: pl.BlockSpec((rr, c), lambda i: (i, 0))
    full = lambda rr, c: pl.BlockSpec((rr, c), lambda i: (0, 0))
    return pl.pallas_call(
        _edge_body,
        grid=(E // BE,),
        in_specs=[
            blk(BE, D), blk(BE, D), blk(BE, D), blk(BE, NUM_FES), blk(BE, 2),
            full(3 * D, H1), full(1, H1),
            full(H1, D_VAL), full(1, D_VAL),
            full(NUM_FES, H_FC),
            full(H_FC, H_FC * D_VAL), full(D_VAL, H_FC * D_VAL),
            full(H_FC * D_VAL, D),
        ],
        out_specs=[blk(BE, D), blk(BE, D)],
        out_shape=[jax.ShapeDtypeStruct((E, D), jnp.float32),
                   jax.ShapeDtypeStruct((E, D), jnp.float32)],
    )(he, hns, hnd, fes, fn, w1, b1, w2, b2, fw1, r, s, w2r)


BN = 1000


def _node_body(hn_r, pr_r, w1a_r, w1b_r, b1_r, w2_r, b2_r, hnn_r):
    nt = pr_r[0] + pr_r[1]
    u = jnp.dot(hn_r[:], w1a_r[:], preferred_element_type=jnp.float32)
    u = u + jnp.dot(nt, w1b_r[:], preferred_element_type=jnp.float32)
    u = jnp.maximum(u + b1_r[:], 0.0)
    hnn_r[:] = hn_r[:] + jnp.dot(u, w2_r[:],
                                 preferred_element_type=jnp.float32) + b2_r[:]


def _node_call(hn, partials, w1a, w1b, b1, w2, b2):
    return pl.pallas_call(
        _node_body,
        grid=(N_NODES // BN,),
        in_specs=[
            pl.BlockSpec((BN, D), lambda i: (i, 0)),
            pl.BlockSpec((NC, BN, D), lambda i: (0, i, 0)),
            pl.BlockSpec((D, H1), lambda i: (0, 0)),
            pl.BlockSpec((D, H1), lambda i: (0, 0)),
            pl.BlockSpec((1, H1), lambda i: (0, 0)),
            pl.BlockSpec((H1, D), lambda i: (0, 0)),
            pl.BlockSpec((1, D), lambda i: (0, 0)),
        ],
        out_specs=pl.BlockSpec((BN, D), lambda i: (i, 0)),
        out_shape=jax.ShapeDtypeStruct((N_NODES, D), jnp.float32),
    )(hn, partials, w1a, w1b, b1, w2, b2)


@jax.jit
def kernel(hn, he, edge_index, fe, fes, norm,
           ev_W1, ev_b1, ev_W2, ev_b2, fc_W1, fc_W2,
           nu_W1, nu_b1, nu_W2, nu_b2):
    src = edge_index[0]
    dst = edge_index[1]
    hns, hnd = _gather(hn, src, dst)
    fn = jnp.concatenate([fe, norm[:, None]], axis=1)
    bf = jnp.bfloat16
    hen, henw = _edge_call(
        he, hns, hnd, fes, fn,
        ev_W1.astype(bf),
        ev_b1.reshape(1, H1), ev_W2.astype(bf), ev_b2.reshape(1, D_VAL),
        fc_W1.astype(bf),
        jnp.kron(jnp.eye(H_FC, dtype=jnp.float32),
                 jnp.ones((1, D_VAL), jnp.float32)).astype(bf),
        jnp.tile(jnp.eye(D_VAL, dtype=jnp.float32), (1, H_FC)).astype(bf),
        fc_W2.reshape(H_FC * D_VAL, D).astype(bf))
    partials = _scatter(henw, dst, jnp.zeros((NR_CHUNK, D), jnp.float32))
    hnn = _node_call(hn, partials,
                     nu_W1[:D], nu_W1[D:], nu_b1.reshape(1, H1),
                     nu_W2, nu_b2.reshape(1, D))
    return hnn, hen


# speedup vs baseline: 1.4814x; 1.0843x over previous
import functools
import jax
import jax.numpy as jnp
from jax import lax
from jax.experimental import pallas as pl
from jax.experimental.pallas import tpu as pltpu
from jax.experimental.pallas import tpu_sc as plsc

N_NODES = 10000
E = 160000
D = 128
D_VAL = 16
NUM_FES = 16
H1 = 512
H_FC = 64
CHUNK = 128
NCHUNKS = E // CHUNK
NC, NS = 2, 16
NW = NC * NS
ITERS = (NCHUNKS + NW - 1) // NW
NR_CHUNK = 80
NRCHUNKS = N_NODES // NR_CHUNK
NR_ITERS = (NRCHUNKS + NS - 1) // NS

_mesh = plsc.VectorSubcoreMesh(core_axis_name="c", subcore_axis_name="s")


def _gather_body(hn_hbm, src_hbm, dst_hbm, hns_hbm, hnd_hbm,
                 idx_s, idx_d, rows_s, rows_d, gsem0, gsem1):
    cid = lax.axis_index("c")
    sid = lax.axis_index("s")
    wid = sid * NC + cid
    gsems = (gsem0, gsem1)

    def chunk_base(step):
        return jnp.minimum(wid + step * NW, NCHUNKS - 1) * CHUNK

    def fire(step, b):
        base = chunk_base(step)
        pltpu.sync_copy(src_hbm.at[pl.ds(base, CHUNK)], idx_s.at[b])
        pltpu.sync_copy(dst_hbm.at[pl.ds(base, CHUNK)], idx_d.at[b])
        ca = pltpu.async_copy(hn_hbm.at[idx_s.at[b]], rows_s.at[b], gsems[b])
        cb = pltpu.async_copy(hn_hbm.at[idx_d.at[b]], rows_d.at[b], gsems[b])
        return ca, cb

    def complete(step, b):
        base = chunk_base(step)
        pltpu.make_async_copy(hn_hbm.at[idx_s.at[b]], rows_s.at[b],
                              gsems[b]).wait()
        pltpu.make_async_copy(hn_hbm.at[idx_d.at[b]], rows_d.at[b],
                              gsems[b]).wait()
        pltpu.sync_copy(rows_s.at[b], hns_hbm.at[pl.ds(base, CHUNK)])
        pltpu.sync_copy(rows_d.at[b], hnd_hbm.at[pl.ds(base, CHUNK)])

    def body(i, carry):
        s0 = 2 * i
        fire(s0, 0)

        @pl.when(i > 0)
        def _():
            complete(s0 - 1, 1)

        fire(s0 + 1, 1)
        complete(s0, 0)
        return carry

    lax.fori_loop(0, ITERS // 2, body, 0)
    complete(ITERS - 1, 1)


_gather = pl.kernel(
    _gather_body,
    mesh=_mesh,
    out_type=[jax.ShapeDtypeStruct((E, D), jnp.float32),
              jax.ShapeDtypeStruct((E, D), jnp.float32)],
    scratch_types=[
        pltpu.VMEM((2, CHUNK), jnp.int32),
        pltpu.VMEM((2, CHUNK), jnp.int32),
        pltpu.VMEM((2, CHUNK, D), jnp.float32),
        pltpu.VMEM((2, CHUNK, D), jnp.float32),
        pltpu.SemaphoreType.DMA,
        pltpu.SemaphoreType.DMA,
    ],
)


def _scatter_body(henw_hbm, dst_hbm, zeros_hbm, out_hbm, idx2, rows,
                  lsem0, lsem1, acc):
    cid = lax.axis_index("c")
    sid = lax.axis_index("s")
    wid = sid * NC + cid
    lsems = (lsem0, lsem1)

    def zbody(i, carry):
        c = sid + i * NS

        @pl.when(c < NRCHUNKS)
        def _():
            pltpu.sync_copy(zeros_hbm, acc.at[pl.ds(c * NR_CHUNK, NR_CHUNK)])

        return carry

    lax.fori_loop(0, NR_ITERS, zbody, 0)
    plsc.subcore_barrier()

    def fire(step, b):
        c = wid + step * NW

        @pl.when(c < NCHUNKS)
        def _():
            base = c * CHUNK
            pltpu.async_copy(dst_hbm.at[pl.ds(base, CHUNK)], idx2.at[b],
                             lsems[b])
            pltpu.async_copy(henw_hbm.at[pl.ds(base, CHUNK)], rows.at[b],
                             lsems[b])

    def complete(step, b):
        c = wid + step * NW

        @pl.when(c < NCHUNKS)
        def _():
            pltpu.make_async_copy(dst_hbm.at[pl.ds(0, CHUNK)], idx2.at[b],
                                  lsems[b]).wait()
            pltpu.make_async_copy(henw_hbm.at[pl.ds(0, CHUNK)], rows.at[b],
                                  lsems[b]).wait()
            pltpu.sync_copy(rows.at[b], acc.at[idx2.at[b]], add=True)

    def body(i, carry):
        s0 = 2 * i
        fire(s0, 0)

        @pl.when(i > 0)
        def _():
            complete(s0 - 1, 1)

        fire(s0 + 1, 1)
        complete(s0, 0)
        return carry

    lax.fori_loop(0, ITERS // 2, body, 0)
    complete(ITERS - 1, 1)
    plsc.subcore_barrier()

    def dbody(i, carry):
        c = sid + i * NS

        @pl.when(c < NRCHUNKS)
        def _():
            pltpu.sync_copy(acc.at[pl.ds(c * NR_CHUNK, NR_CHUNK)],
                            out_hbm.at[cid, pl.ds(c * NR_CHUNK, NR_CHUNK)])

        return carry

    lax.fori_loop(0, NR_ITERS, dbody, 0)


_scatter = pl.kernel(
    _scatter_body,
    mesh=_mesh,
    out_type=jax.ShapeDtypeStruct((NC, N_NODES, D), jnp.float32),
    scratch_types=[
        pltpu.VMEM((2, CHUNK), jnp.int32),
        pltpu.VMEM((2, CHUNK, D), jnp.float32),
        pltpu.SemaphoreType.DMA,
        pltpu.SemaphoreType.DMA,
        pltpu.VMEM_SHARED((N_NODES, D), jnp.float32),
    ],
)


BE = 640


def _edge_body(he_r, hns_r, hnd_r, fes_r, fn_r,
               w1_r, b1_r, w2_r, b2_r, fw1_r, r_r, s_r, w2r_r,
               hen_r, henw_r):
    bf = jnp.bfloat16
    z = jnp.concatenate(
        [he_r[:].astype(bf), hns_r[:].astype(bf), hnd_r[:].astype(bf)], axis=1)
    t = jnp.dot(z, w1_r[:], preferred_element_type=jnp.float32)
    t = jnp.maximum(t + b1_r[:], 0.0).astype(bf)
    v = jnp.dot(t, w2_r[:], preferred_element_type=jnp.float32) + b2_r[:]
    h = jnp.maximum(
        jnp.dot(fes_r[:].astype(bf), fw1_r[:],
                preferred_element_type=jnp.float32) * 0.25,
        0.0).astype(bf)
    v_s = (v * (fn_r[:, 0:1] * (1.0 / 32.0))).astype(bf)
    h_rep = jnp.dot(h, r_r[:], preferred_element_type=jnp.float32)
    v_tile = jnp.dot(v_s, s_r[:], preferred_element_type=jnp.float32)
    p = (h_rep * v_tile).astype(bf)
    heu = jnp.dot(p, w2r_r[:], preferred_element_type=jnp.float32)
    hen = he_r[:] + heu
    hen_r[:] = hen
    henw_r[:] = hen * fn_r[:, 1:2]


def _edge_call(he, hns, hnd, fes, fn, w1, b1, w2, b2, fw1, r, s, w2r):
    blk = lambda rr, c: pl.BlockSpec((rr, c), lambda i: (i, 0))
    full = lambda rr, c: pl.BlockSpec((rr, c), lambda i: (0, 0))
    return pl.pallas_call(
        _edge_body,
        grid=(E // BE,),
        in_specs=[
            blk(BE, D), blk(BE, D), blk(BE, D), blk(BE, NUM_FES), blk(BE, 2),
            full(3 * D, H1), full(1, H1),
            full(H1, D_VAL), full(1, D_VAL),
            full(NUM_FES, H_FC),
            full(H_FC, H_FC * D_VAL), full(D_VAL, H_FC * D_VAL),
            full(H_FC * D_VAL, D),
        ],
        out_specs=[blk(BE, D), blk(BE, D)],
        out_shape=[jax.ShapeDtypeStruct((E, D), jnp.float32),
                   jax.ShapeDtypeStruct((E, D), jnp.float32)],
    )(he, hns, hnd, fes, fn, w1, b1, w2, b2, fw1, r, s, w2r)


BN = 1000


def _node_body(hn_r, pr_r, w1a_r, w1b_r, b1_r, w2_r, b2_r, hnn_r):
    nt = pr_r[0] + pr_r[1]
    u = jnp.dot(hn_r[:], w1a_r[:], preferred_element_type=jnp.float32)
    u = u + jnp.dot(nt, w1b_r[:], preferred_element_type=jnp.float32)
    u = jnp.maximum(u + b1_r[:], 0.0)
    hnn_r[:] = hn_r[:] + jnp.dot(u, w2_r[:],
                                 preferred_element_type=jnp.float32) + b2_r[:]


def _node_call(hn, partials, w1a, w1b, b1, w2, b2):
    return pl.pallas_call(
        _node_body,
        grid=(N_NODES // BN,),
        in_specs=[
            pl.BlockSpec((BN, D), lambda i: (i, 0)),
            pl.BlockSpec((NC, BN, D), lambda i: (0, i, 0)),
            pl.BlockSpec((D, H1), lambda i: (0, 0)),
            pl.BlockSpec((D, H1), lambda i: (0, 0)),
            pl.BlockSpec((1, H1), lambda i: (0, 0)),
            pl.BlockSpec((H1, D), lambda i: (0, 0)),
            pl.BlockSpec((1, D), lambda i: (0, 0)),
        ],
        out_specs=pl.BlockSpec((BN, D), lambda i: (i, 0)),
        out_shape=jax.ShapeDtypeStruct((N_NODES, D), jnp.float32),
    )(hn, partials, w1a, w1b, b1, w2, b2)


@jax.jit
def kernel(hn, he, edge_index, fe, fes, norm,
           ev_W1, ev_b1, ev_W2, ev_b2, fc_W1, fc_W2,
           nu_W1, nu_b1, nu_W2, nu_b2):
    src = edge_index[0]
    dst = edge_index[1]
    hns, hnd = _gather(hn, src, dst)
    fn = jnp.concatenate([fe, norm[:, None]], axis=1)
    bf = jnp.bfloat16
    hen, henw = _edge_call(
        he, hns, hnd, fes, fn,
        ev_W1.astype(bf),
        ev_b1.reshape(1, H1), ev_W2.astype(bf), ev_b2.reshape(1, D_VAL),
        fc_W1.astype(bf),
        jnp.kron(jnp.eye(H_FC, dtype=jnp.float32),
                 jnp.ones((1, D_VAL), jnp.float32)).astype(bf),
        jnp.tile(jnp.eye(D_VAL, dtype=jnp.float32), (1, H_FC)).astype(bf),
        fc_W2.reshape(H_FC * D_VAL, D).astype(bf))
    partials = _scatter(henw, dst, jnp.zeros((NR_CHUNK, D), jnp.float32))
    hnn = _node_call(hn, partials,
                     nu_W1[:D], nu_W1[D:], nu_b1.reshape(1, H1),
                     nu_W2, nu_b2.reshape(1, D))
    return hnn, hen


# speedup vs baseline: 1.6220x; 1.0949x over previous
import functools
import jax
import jax.numpy as jnp
from jax import lax
from jax.experimental import pallas as pl
from jax.experimental.pallas import tpu as pltpu
from jax.experimental.pallas import tpu_sc as plsc

N_NODES = 10000
E = 160000
D = 128
D_VAL = 16
NUM_FES = 16
H1 = 512
H_FC = 64
CHUNK = 128
NCHUNKS = E // CHUNK
NC, NS = 2, 16
NW = NC * NS
ITERS = (NCHUNKS + NW - 1) // NW
NR_CHUNK = 80
NRCHUNKS = N_NODES // NR_CHUNK
NR_ITERS = (NRCHUNKS + NS - 1) // NS

_mesh = plsc.VectorSubcoreMesh(core_axis_name="c", subcore_axis_name="s")


def _gather_body(hn_hbm, src_hbm, dst_hbm, hns_hbm, hnd_hbm,
                 idx_s, idx_d, rows_s, rows_d, gsem0, gsem1):
    cid = lax.axis_index("c")
    sid = lax.axis_index("s")
    wid = sid * NC + cid
    gsems = (gsem0, gsem1)

    def chunk_base(step):
        return jnp.minimum(wid + step * NW, NCHUNKS - 1) * CHUNK

    def fire(step, b):
        base = chunk_base(step)
        pltpu.sync_copy(src_hbm.at[pl.ds(base, CHUNK)], idx_s.at[b])
        pltpu.sync_copy(dst_hbm.at[pl.ds(base, CHUNK)], idx_d.at[b])
        ca = pltpu.async_copy(hn_hbm.at[idx_s.at[b]], rows_s.at[b], gsems[b])
        cb = pltpu.async_copy(hn_hbm.at[idx_d.at[b]], rows_d.at[b], gsems[b])
        return ca, cb

    def complete(step, b):
        base = chunk_base(step)
        pltpu.make_async_copy(hn_hbm.at[idx_s.at[b]], rows_s.at[b],
                              gsems[b]).wait()
        pltpu.make_async_copy(hn_hbm.at[idx_d.at[b]], rows_d.at[b],
                              gsems[b]).wait()
        pltpu.sync_copy(rows_s.at[b], hns_hbm.at[pl.ds(base, CHUNK)])
        pltpu.sync_copy(rows_d.at[b], hnd_hbm.at[pl.ds(base, CHUNK)])

    def body(i, carry):
        s0 = 2 * i
        fire(s0, 0)

        @pl.when(i > 0)
        def _():
            complete(s0 - 1, 1)

        fire(s0 + 1, 1)
        complete(s0, 0)
        return carry

    lax.fori_loop(0, ITERS // 2, body, 0)
    complete(ITERS - 1, 1)


_gather = pl.kernel(
    _gather_body,
    mesh=_mesh,
    out_type=[jax.ShapeDtypeStruct((E, D), jnp.float32),
              jax.ShapeDtypeStruct((E, D), jnp.float32)],
    scratch_types=[
        pltpu.VMEM((2, CHUNK), jnp.int32),
        pltpu.VMEM((2, CHUNK), jnp.int32),
        pltpu.VMEM((2, CHUNK, D), jnp.float32),
        pltpu.VMEM((2, CHUNK, D), jnp.float32),
        pltpu.SemaphoreType.DMA,
        pltpu.SemaphoreType.DMA,
    ],
)


def _scatter_body(henw_hbm, dst_hbm, zeros_hbm, out_hbm, idx2, rows,
                  lsem0, lsem1, acc):
    cid = lax.axis_index("c")
    sid = lax.axis_index("s")
    wid = sid * NC + cid
    lsems = (lsem0, lsem1)

    def zbody(i, carry):
        c = sid + i * NS

        @pl.when(c < NRCHUNKS)
        def _():
            pltpu.sync_copy(zeros_hbm, acc.at[pl.ds(c * NR_CHUNK, NR_CHUNK)])

        return carry

    lax.fori_loop(0, NR_ITERS, zbody, 0)
    plsc.subcore_barrier()

    def fire(step, b):
        c = wid + step * NW

        @pl.when(c < NCHUNKS)
        def _():
            base = c * CHUNK
            pltpu.async_copy(dst_hbm.at[pl.ds(base, CHUNK)], idx2.at[b],
                             lsems[b])
            pltpu.async_copy(henw_hbm.at[pl.ds(base, CHUNK)], rows.at[b],
                             lsems[b])

    def complete(step, b):
        c = wid + step * NW

        @pl.when(c < NCHUNKS)
        def _():
            pltpu.make_async_copy(dst_hbm.at[pl.ds(0, CHUNK)], idx2.at[b],
                                  lsems[b]).wait()
            pltpu.make_async_copy(henw_hbm.at[pl.ds(0, CHUNK)], rows.at[b],
                                  lsems[b]).wait()
            pltpu.sync_copy(rows.at[b], acc.at[idx2.at[b]], add=True)

    def body(i, carry):
        s0 = 2 * i
        fire(s0, 0)

        @pl.when(i > 0)
        def _():
            complete(s0 - 1, 1)

        fire(s0 + 1, 1)
        complete(s0, 0)
        return carry

    lax.fori_loop(0, ITERS // 2, body, 0)
    complete(ITERS - 1, 1)
    plsc.subcore_barrier()

    def dbody(i, carry):
        c = sid + i * NS

        @pl.when(c < NRCHUNKS)
        def _():
            pltpu.sync_copy(acc.at[pl.ds(c * NR_CHUNK, NR_CHUNK)],
                            out_hbm.at[cid, pl.ds(c * NR_CHUNK, NR_CHUNK)])

        return carry

    lax.fori_loop(0, NR_ITERS, dbody, 0)


_scatter = pl.kernel(
    _scatter_body,
    mesh=_mesh,
    out_type=jax.ShapeDtypeStruct((NC, N_NODES, D), jnp.float32),
    scratch_types=[
        pltpu.VMEM((2, CHUNK), jnp.int32),
        pltpu.VMEM((2, CHUNK, D), jnp.float32),
        pltpu.SemaphoreType.DMA,
        pltpu.SemaphoreType.DMA,
        pltpu.VMEM_SHARED((N_NODES, D), jnp.float32),
    ],
)


BE = 1600


def _edge_body(he_r, hns_r, hnd_r, fes_r, fn_r,
               w1_r, b1_r, w2_r, b2_r, fw1_r, r_r, s_r, w2r_r,
               hen_r, henw_r):
    bf = jnp.bfloat16
    z = jnp.concatenate(
        [he_r[:].astype(bf), hns_r[:].astype(bf), hnd_r[:].astype(bf)], axis=1)
    t = jnp.dot(z, w1_r[:], preferred_element_type=jnp.float32)
    t = jnp.maximum(t + b1_r[:], 0.0).astype(bf)
    v = jnp.dot(t, w2_r[:], preferred_element_type=jnp.float32) + b2_r[:]
    h = jnp.maximum(
        jnp.dot(fes_r[:].astype(bf), fw1_r[:],
                preferred_element_type=jnp.float32) * 0.25,
        0.0).astype(bf)
    v_s = (v * (fn_r[:, 0:1] * (1.0 / 32.0))).astype(bf)
    h_rep = jnp.dot(h, r_r[:], preferred_element_type=jnp.float32)
    v_tile = jnp.dot(v_s, s_r[:], preferred_element_type=jnp.float32)
    p = (h_rep * v_tile).astype(bf)
    heu = jnp.dot(p, w2r_r[:], preferred_element_type=jnp.float32)
    hen = he_r[:] + heu
    hen_r[:] = hen
    henw_r[:] = hen * fn_r[:, 1:2]


def _edge_call(he, hns, hnd, fes, fn, w1, b1, w2, b2, fw1, r, s, w2r):
    blk = lambda rr, c: pl.BlockSpec((rr, c), lambda i: (i, 0))
    full = lambda rr, c: pl.BlockSpec((rr, c), lambda i: (0, 0))
    return pl.pallas_call(
        _edge_body,
        grid=(E // BE,),
        in_specs=[
            blk(BE, D), blk(BE, D), blk(BE, D), blk(BE, NUM_FES), blk(BE, 2),
            full(3 * D, H1), full(1, H1),
            full(H1, D_VAL), full(1, D_VAL),
            full(NUM_FES, H_FC),
            full(H_FC, H_FC * D_VAL), full(D_VAL, H_FC * D_VAL),
            full(H_FC * D_VAL, D),
        ],
        out_specs=[blk(BE, D), blk(BE, D)],
        out_shape=[jax.ShapeDtypeStruct((E, D), jnp.float32),
                   jax.ShapeDtypeStruct((E, D), jnp.float32)],
    )(he, hns, hnd, fes, fn, w1, b1, w2, b2, fw1, r, s, w2r)


BN = 1000


def _node_body(hn_r, pr_r, w1a_r, w1b_r, b1_r, w2_r, b2_r, hnn_r):
    nt = pr_r[0] + pr_r[1]
    u = jnp.dot(hn_r[:], w1a_r[:], preferred_element_type=jnp.float32)
    u = u + jnp.dot(nt, w1b_r[:], preferred_element_type=jnp.float32)
    u = jnp.maximum(u + b1_r[:], 0.0)
    hnn_r[:] = hn_r[:] + jnp.dot(u, w2_r[:],
                                 preferred_element_type=jnp.float32) + b2_r[:]


def _node_call(hn, partials, w1a, w1b, b1, w2, b2):
    return pl.pallas_call(
        _node_body,
        grid=(N_NODES // BN,),
        in_specs=[
            pl.BlockSpec((BN, D), lambda i: (i, 0)),
            pl.BlockSpec((NC, BN, D), lambda i: (0, i, 0)),
            pl.BlockSpec((D, H1), lambda i: (0, 0)),
            pl.BlockSpec((D, H1), lambda i: (0, 0)),
            pl.BlockSpec((1, H1), lambda i: (0, 0)),
            pl.BlockSpec((H1, D), lambda i: (0, 0)),
            pl.BlockSpec((1, D), lambda i: (0, 0)),
        ],
        out_specs=pl.BlockSpec((BN, D), lambda i: (i, 0)),
        out_shape=jax.ShapeDtypeStruct((N_NODES, D), jnp.float32),
    )(hn, partials, w1a, w1b, b1, w2, b2)


@jax.jit
def kernel(hn, he, edge_index, fe, fes, norm,
           ev_W1, ev_b1, ev_W2, ev_b2, fc_W1, fc_W2,
           nu_W1, nu_b1, nu_W2, nu_b2):
    src = edge_index[0]
    dst = edge_index[1]
    hns, hnd = _gather(hn, src, dst)
    fn = jnp.concatenate([fe, norm[:, None]], axis=1)
    bf = jnp.bfloat16
    hen, henw = _edge_call(
        he, hns, hnd, fes, fn,
        ev_W1.astype(bf),
        ev_b1.reshape(1, H1), ev_W2.astype(bf), ev_b2.reshape(1, D_VAL),
        fc_W1.astype(bf),
        jnp.kron(jnp.eye(H_FC, dtype=jnp.float32),
                 jnp.ones((1, D_VAL), jnp.float32)).astype(bf),
        jnp.tile(jnp.eye(D_VAL, dtype=jnp.float32), (1, H_FC)).astype(bf),
        fc_W2.reshape(H_FC * D_VAL, D).astype(bf))
    partials = _scatter(henw, dst, jnp.zeros((NR_CHUNK, D), jnp.float32))
    hnn = _node_call(hn, partials,
                     nu_W1[:D], nu_W1[D:], nu_b1.reshape(1, H1),
                     nu_W2, nu_b2.reshape(1, D))
    return hnn, hen


# speedup vs baseline: 1.6629x; 1.0252x over previous
import functools
import jax
import jax.numpy as jnp
from jax import lax
from jax.experimental import pallas as pl
from jax.experimental.pallas import tpu as pltpu
from jax.experimental.pallas import tpu_sc as plsc

N_NODES = 10000
E = 160000
D = 128
D_VAL = 16
NUM_FES = 16
H1 = 512
H_FC = 64
CHUNK = 128
NCHUNKS = E // CHUNK
NC, NS = 2, 16
NW = NC * NS
ITERS = (NCHUNKS + NW - 1) // NW
NR_CHUNK = 80
NRCHUNKS = N_NODES // NR_CHUNK
NR_ITERS = (NRCHUNKS + NS - 1) // NS

_mesh = plsc.VectorSubcoreMesh(core_axis_name="c", subcore_axis_name="s")


def _gather_body(hn_hbm, src_hbm, dst_hbm, hns_hbm, hnd_hbm,
                 idx_s, idx_d, rows_s, rows_d, gsem0, gsem1):
    cid = lax.axis_index("c")
    sid = lax.axis_index("s")
    wid = sid * NC + cid
    gsems = (gsem0, gsem1)

    def chunk_base(step):
        return jnp.minimum(wid + step * NW, NCHUNKS - 1) * CHUNK

    def fire(step, b):
        base = chunk_base(step)
        pltpu.sync_copy(src_hbm.at[pl.ds(base, CHUNK)], idx_s.at[b])
        pltpu.sync_copy(dst_hbm.at[pl.ds(base, CHUNK)], idx_d.at[b])
        ca = pltpu.async_copy(hn_hbm.at[idx_s.at[b]], rows_s.at[b], gsems[b])
        cb = pltpu.async_copy(hn_hbm.at[idx_d.at[b]], rows_d.at[b], gsems[b])
        return ca, cb

    def complete(step, b):
        base = chunk_base(step)
        pltpu.make_async_copy(hn_hbm.at[idx_s.at[b]], rows_s.at[b],
                              gsems[b]).wait()
        pltpu.make_async_copy(hn_hbm.at[idx_d.at[b]], rows_d.at[b],
                              gsems[b]).wait()
        pltpu.sync_copy(rows_s.at[b], hns_hbm.at[pl.ds(base, CHUNK)])
        pltpu.sync_copy(rows_d.at[b], hnd_hbm.at[pl.ds(base, CHUNK)])

    def body(i, carry):
        s0 = 2 * i
        fire(s0, 0)

        @pl.when(i > 0)
        def _():
            complete(s0 - 1, 1)

        fire(s0 + 1, 1)
        complete(s0, 0)
        return carry

    lax.fori_loop(0, ITERS // 2, body, 0)
    complete(ITERS - 1, 1)


_gather = pl.kernel(
    _gather_body,
    mesh=_mesh,
    out_type=[jax.ShapeDtypeStruct((E, D), jnp.float32),
              jax.ShapeDtypeStruct((E, D), jnp.float32)],
    scratch_types=[
        pltpu.VMEM((2, CHUNK), jnp.int32),
        pltpu.VMEM((2, CHUNK), jnp.int32),
        pltpu.VMEM((2, CHUNK, D), jnp.float32),
        pltpu.VMEM((2, CHUNK, D), jnp.float32),
        pltpu.SemaphoreType.DMA,
        pltpu.SemaphoreType.DMA,
    ],
)


def _scatter_body(henw_hbm, dst_hbm, zeros_hbm, out_hbm, idx2, rows,
                  lsem0, lsem1, acc):
    cid = lax.axis_index("c")
    sid = lax.axis_index("s")
    wid = sid * NC + cid
    lsems = (lsem0, lsem1)

    def zbody(i, carry):
        c = sid + i * NS

        @pl.when(c < NRCHUNKS)
        def _():
            pltpu.sync_copy(zeros_hbm, acc.at[pl.ds(c * NR_CHUNK, NR_CHUNK)])

        return carry

    lax.fori_loop(0, NR_ITERS, zbody, 0)
    plsc.subcore_barrier()

    def fire(step, b):
        c = wid + step * NW

        @pl.when(c < NCHUNKS)
        def _():
            base = c * CHUNK
            pltpu.async_copy(dst_hbm.at[pl.ds(base, CHUNK)], idx2.at[b],
                             lsems[b])
            pltpu.async_copy(henw_hbm.at[pl.ds(base, CHUNK)], rows.at[b],
                             lsems[b])

    def complete(step, b):
        c = wid + step * NW

        @pl.when(c < NCHUNKS)
        def _():
            pltpu.make_async_copy(dst_hbm.at[pl.ds(0, CHUNK)], idx2.at[b],
                                  lsems[b]).wait()
            pltpu.make_async_copy(henw_hbm.at[pl.ds(0, CHUNK)], rows.at[b],
                                  lsems[b]).wait()
            pltpu.sync_copy(rows.at[b], acc.at[idx2.at[b]], add=True)

    def body(i, carry):
        s0 = 2 * i
        fire(s0, 0)

        @pl.when(i > 0)
        def _():
            complete(s0 - 1, 1)

        fire(s0 + 1, 1)
        complete(s0, 0)
        return carry

    lax.fori_loop(0, ITERS // 2, body, 0)
    complete(ITERS - 1, 1)
    plsc.subcore_barrier()

    def dbody(i, carry):
        c = sid + i * NS

        @pl.when(c < NRCHUNKS)
        def _():
            pltpu.sync_copy(acc.at[pl.ds(c * NR_CHUNK, NR_CHUNK)],
                            out_hbm.at[cid, pl.ds(c * NR_CHUNK, NR_CHUNK)])

        return carry

    lax.fori_loop(0, NR_ITERS, dbody, 0)


_scatter = pl.kernel(
    _scatter_body,
    mesh=_mesh,
    out_type=jax.ShapeDtypeStruct((NC, N_NODES, D), jnp.float32),
    scratch_types=[
        pltpu.VMEM((2, CHUNK), jnp.int32),
        pltpu.VMEM((2, CHUNK, D), jnp.float32),
        pltpu.SemaphoreType.DMA,
        pltpu.SemaphoreType.DMA,
        pltpu.VMEM_SHARED((N_NODES, D), jnp.float32),
    ],
)


BE = 3200


def _edge_body(he_r, hns_r, hnd_r, fes_r, fn_r,
               w1_r, b1_r, w2_r, b2_r, fw1_r, r_r, s_r, w2r_r,
               hen_r, henw_r):
    bf = jnp.bfloat16
    z = jnp.concatenate(
        [he_r[:].astype(bf), hns_r[:].astype(bf), hnd_r[:].astype(bf)], axis=1)
    t = jnp.dot(z, w1_r[:], preferred_element_type=jnp.float32)
    t = jnp.maximum(t + b1_r[:], 0.0).astype(bf)
    v = jnp.dot(t, w2_r[:], preferred_element_type=jnp.float32) + b2_r[:]
    h = jnp.maximum(
        jnp.dot(fes_r[:].astype(bf), fw1_r[:],
                preferred_element_type=jnp.float32) * 0.25,
        0.0).astype(bf)
    v_s = (v * (fn_r[:, 0:1] * (1.0 / 32.0))).astype(bf)
    h_rep = jnp.dot(h, r_r[:], preferred_element_type=jnp.float32)
    v_tile = jnp.dot(v_s, s_r[:], preferred_element_type=jnp.float32)
    p = (h_rep * v_tile).astype(bf)
    heu = jnp.dot(p, w2r_r[:], preferred_element_type=jnp.float32)
    hen = he_r[:] + heu
    hen_r[:] = hen
    henw_r[:] = hen * fn_r[:, 1:2]


def _edge_call(he, hns, hnd, fes, fn, w1, b1, w2, b2, fw1, r, s, w2r):
    blk = lambda rr, c: pl.BlockSpec((rr, c), lambda i: (i, 0))
    full = lambda rr, c: pl.BlockSpec((rr, c), lambda i: (0, 0))
    return pl.pallas_call(
        _edge_body,
        grid=(E // BE,),
        in_specs=[
            blk(BE, D), blk(BE, D), blk(BE, D), blk(BE, NUM_FES), blk(BE, 2),
            full(3 * D, H1), full(1, H1),
            full(H1, D_VAL), full(1, D_VAL),
            full(NUM_FES, H_FC),
            full(H_FC, H_FC * D_VAL), full(D_VAL, H_FC * D_VAL),
            full(H_FC * D_VAL, D),
        ],
        out_specs=[blk(BE, D), blk(BE, D)],
        out_shape=[jax.ShapeDtypeStruct((E, D), jnp.float32),
                   jax.ShapeDtypeStruct((E, D), jnp.float32)],
    )(he, hns, hnd, fes, fn, w1, b1, w2, b2, fw1, r, s, w2r)


BN = 1000


def _node_body(hn_r, pr_r, w1a_r, w1b_r, b1_r, w2_r, b2_r, hnn_r):
    nt = pr_r[0] + pr_r[1]
    u = jnp.dot(hn_r[:], w1a_r[:], preferred_element_type=jnp.float32)
    u = u + jnp.dot(nt, w1b_r[:], preferred_element_type=jnp.float32)
    u = jnp.maximum(u + b1_r[:], 0.0)
    hnn_r[:] = hn_r[:] + jnp.dot(u, w2_r[:],
                                 preferred_element_type=jnp.float32) + b2_r[:]


def _node_call(hn, partials, w1a, w1b, b1, w2, b2):
    return pl.pallas_call(
        _node_body,
        grid=(N_NODES // BN,),
        in_specs=[
            pl.BlockSpec((BN, D), lambda i: (i, 0)),
            pl.BlockSpec((NC, BN, D), lambda i: (0, i, 0)),
            pl.BlockSpec((D, H1), lambda i: (0, 0)),
            pl.BlockSpec((D, H1), lambda i: (0, 0)),
            pl.BlockSpec((1, H1), lambda i: (0, 0)),
            pl.BlockSpec((H1, D), lambda i: (0, 0)),
            pl.BlockSpec((1, D), lambda i: (0, 0)),
        ],
        out_specs=pl.BlockSpec((BN, D), lambda i: (i, 0)),
        out_shape=jax.ShapeDtypeStruct((N_NODES, D), jnp.float32),
    )(hn, partials, w1a, w1b, b1, w2, b2)


@jax.jit
def kernel(hn, he, edge_index, fe, fes, norm,
           ev_W1, ev_b1, ev_W2, ev_b2, fc_W1, fc_W2,
           nu_W1, nu_b1, nu_W2, nu_b2):
    src = edge_index[0]
    dst = edge_index[1]
    hns, hnd = _gather(hn, src, dst)
    fn = jnp.concatenate([fe, norm[:, None]], axis=1)
    bf = jnp.bfloat16
    hen, henw = _edge_call(
        he, hns, hnd, fes, fn,
        ev_W1.astype(bf),
        ev_b1.reshape(1, H1), ev_W2.astype(bf), ev_b2.reshape(1, D_VAL),
        fc_W1.astype(bf),
        jnp.kron(jnp.eye(H_FC, dtype=jnp.float32),
                 jnp.ones((1, D_VAL), jnp.float32)).astype(bf),
        jnp.tile(jnp.eye(D_VAL, dtype=jnp.float32), (1, H_FC)).astype(bf),
        fc_W2.reshape(H_FC * D_VAL, D).astype(bf))
    partials = _scatter(henw, dst, jnp.zeros((NR_CHUNK, D), jnp.float32))
    hnn = _node_call(hn, partials,
                     nu_W1[:D], nu_W1[D:], nu_b1.reshape(1, H1),
                     nu_W2, nu_b2.reshape(1, D))
    return hnn, hen


# speedup vs baseline: 1.6821x; 1.0115x over previous
import functools
import jax
import jax.numpy as jnp
from jax import lax
from jax.experimental import pallas as pl
from jax.experimental.pallas import tpu as pltpu
from jax.experimental.pallas import tpu_sc as plsc

N_NODES = 10000
E = 160000
D = 128
D_VAL = 16
NUM_FES = 16
H1 = 512
H_FC = 64
CHUNK = 128
NC, NS = 2, 16
NW = NC * NS
NSPLIT = 2
EH = E // NSPLIT
NCH = EH // CHUNK
ITERS_H = (NCH + NW - 1) // NW
NR_CHUNK = 80
NRCHUNKS = N_NODES // NR_CHUNK
NR_ITERS = (NRCHUNKS + NS - 1) // NS

_mesh = plsc.VectorSubcoreMesh(core_axis_name="c", subcore_axis_name="s")


def _gather_body(chunk_lo, hn_hbm, src_hbm, dst_hbm, hns_hbm, hnd_hbm,
                 idx_s, idx_d, rows_s, rows_d, gsem0, gsem1):
    cid = lax.axis_index("c")
    sid = lax.axis_index("s")
    wid = sid * NC + cid
    gsems = (gsem0, gsem1)

    def chunk(step):
        return jnp.minimum(wid + step * NW, NCH - 1)

    def fire(step, b):
        base = (chunk(step) + chunk_lo) * CHUNK
        pltpu.sync_copy(src_hbm.at[pl.ds(base, CHUNK)], idx_s.at[b])
        pltpu.sync_copy(dst_hbm.at[pl.ds(base, CHUNK)], idx_d.at[b])
        pltpu.async_copy(hn_hbm.at[idx_s.at[b]], rows_s.at[b], gsems[b])
        pltpu.async_copy(hn_hbm.at[idx_d.at[b]], rows_d.at[b], gsems[b])

    def complete(step, b):
        base = chunk(step) * CHUNK
        pltpu.make_async_copy(hn_hbm.at[idx_s.at[b]], rows_s.at[b],
                              gsems[b]).wait()
        pltpu.make_async_copy(hn_hbm.at[idx_d.at[b]], rows_d.at[b],
                              gsems[b]).wait()
        pltpu.sync_copy(rows_s.at[b], hns_hbm.at[pl.ds(base, CHUNK)])
        pltpu.sync_copy(rows_d.at[b], hnd_hbm.at[pl.ds(base, CHUNK)])

    def body(i, carry):
        s0 = 2 * i
        fire(s0, 0)

        @pl.when(i > 0)
        def _():
            complete(s0 - 1, 1)

        fire(s0 + 1, 1)
        complete(s0, 0)
        return carry

    lax.fori_loop(0, ITERS_H // 2, body, 0)
    complete(ITERS_H - 1, 1)


def _make_gather(chunk_lo):
    return pl.kernel(
        functools.partial(_gather_body, chunk_lo),
        mesh=_mesh,
        out_type=[jax.ShapeDtypeStruct((EH, D), jnp.float32),
                  jax.ShapeDtypeStruct((EH, D), jnp.float32)],
        scratch_types=[
            pltpu.VMEM((2, CHUNK), jnp.int32),
            pltpu.VMEM((2, CHUNK), jnp.int32),
            pltpu.VMEM((2, CHUNK, D), jnp.float32),
            pltpu.VMEM((2, CHUNK, D), jnp.float32),
            pltpu.SemaphoreType.DMA,
            pltpu.SemaphoreType.DMA,
        ],
    )


_gathers = [_make_gather(k * NCH) for k in range(NSPLIT)]


def _scatter_body(chunk_lo, henw_hbm, dst_hbm, zeros_hbm, out_hbm, idx2, rows,
                  lsem0, lsem1, acc):
    cid = lax.axis_index("c")
    sid = lax.axis_index("s")
    wid = sid * NC + cid
    lsems = (lsem0, lsem1)

    def zbody(i, carry):
        c = sid + i * NS

        @pl.when(c < NRCHUNKS)
        def _():
            pltpu.sync_copy(zeros_hbm, acc.at[pl.ds(c * NR_CHUNK, NR_CHUNK)])

        return carry

    lax.fori_loop(0, NR_ITERS, zbody, 0)
    plsc.subcore_barrier()

    def fire(step, b):
        c = wid + step * NW

        @pl.when(c < NCH)
        def _():
            pltpu.async_copy(dst_hbm.at[pl.ds((c + chunk_lo) * CHUNK, CHUNK)],
                             idx2.at[b], lsems[b])
            pltpu.async_copy(henw_hbm.at[pl.ds(c * CHUNK, CHUNK)],
                             rows.at[b], lsems[b])

    def complete(step, b):
        c = wid + step * NW

        @pl.when(c < NCH)
        def _():
            pltpu.make_async_copy(dst_hbm.at[pl.ds(0, CHUNK)], idx2.at[b],
                                  lsems[b]).wait()
            pltpu.make_async_copy(henw_hbm.at[pl.ds(0, CHUNK)], rows.at[b],
                                  lsems[b]).wait()
            pltpu.sync_copy(rows.at[b], acc.at[idx2.at[b]], add=True)

    def body(i, carry):
        s0 = 2 * i
        fire(s0, 0)

        @pl.when(i > 0)
        def _():
            complete(s0 - 1, 1)

        fire(s0 + 1, 1)
        complete(s0, 0)
        return carry

    lax.fori_loop(0, ITERS_H // 2, body, 0)
    complete(ITERS_H - 1, 1)
    plsc.subcore_barrier()

    def dbody(i, carry):
        c = sid + i * NS

        @pl.when(c < NRCHUNKS)
        def _():
            pltpu.sync_copy(acc.at[pl.ds(c * NR_CHUNK, NR_CHUNK)],
                            out_hbm.at[cid, pl.ds(c * NR_CHUNK, NR_CHUNK)])

        return carry

    lax.fori_loop(0, NR_ITERS, dbody, 0)


def _make_scatter(chunk_lo):
    return pl.kernel(
        functools.partial(_scatter_body, chunk_lo),
        mesh=_mesh,
        out_type=jax.ShapeDtypeStruct((NC, N_NODES, D), jnp.float32),
        scratch_types=[
            pltpu.VMEM((2, CHUNK), jnp.int32),
            pltpu.VMEM((2, CHUNK, D), jnp.float32),
            pltpu.SemaphoreType.DMA,
            pltpu.SemaphoreType.DMA,
            pltpu.VMEM_SHARED((N_NODES, D), jnp.float32),
        ],
    )


_scatters = [_make_scatter(k * NCH) for k in range(NSPLIT)]


BE = 3200
NBLK_H = EH // BE


def _edge_math(he, hns, hnd, fes, fn, w1, b1, w2, b2, fw1, r, s, w2r):
    bf = jnp.bfloat16
    z = jnp.concatenate(
        [he.astype(bf), hns.astype(bf), hnd.astype(bf)], axis=1)
    t = jnp.dot(z, w1, preferred_element_type=jnp.float32)
    t = jnp.maximum(t + b1, 0.0).astype(bf)
    v = jnp.dot(t, w2, preferred_element_type=jnp.float32) + b2
    h = jnp.maximum(
        jnp.dot(fes.astype(bf), fw1,
                preferred_element_type=jnp.float32) * 0.25,
        0.0).astype(bf)
    v_s = (v * (fn[:, 0:1] * (1.0 / 32.0))).astype(bf)
    h_rep = jnp.dot(h, r, preferred_element_type=jnp.float32)
    v_tile = jnp.dot(v_s, s, preferred_element_type=jnp.float32)
    p = (h_rep * v_tile).astype(bf)
    heu = jnp.dot(p, w2r, preferred_element_type=jnp.float32)
    hen = he + heu
    return hen, hen * fn[:, 1:2]


def _edge_body0(he_r, hns_r, hnd_r, fes_r, fn_r,
                w1_r, b1_r, w2_r, b2_r, fw1_r, r_r, s_r, w2r_r,
                hen_r, henw_r):
    hen, henw = _edge_math(he_r[:], hns_r[:], hnd_r[:], fes_r[:], fn_r[:],
                           w1_r[:], b1_r[:], w2_r[:], b2_r[:], fw1_r[:],
                           r_r[:], s_r[:], w2r_r[:])
    hen_r[:] = hen
    henw_r[:] = henw


def _edge_body1(he_r, hns_r, hnd_r, fes_r, fn_r,
                w1_r, b1_r, w2_r, b2_r, fw1_r, r_r, s_r, w2r_r, prev_r,
                hen_r, henw_r):
    hen, henw = _edge_math(he_r[:], hns_r[:], hnd_r[:], fes_r[:], fn_r[:],
                           w1_r[:], b1_r[:], w2_r[:], b2_r[:], fw1_r[:],
                           r_r[:], s_r[:], w2r_r[:])
    hen_r[:] = hen
    henw_r[:] = henw


def _edge_call(half, he, hns, hnd, fes, fn, w1, b1, w2, b2, fw1, r, s, w2r,
               hen_prev=None):
    off = half * NBLK_H
    blk_g = lambda rr, c: pl.BlockSpec((rr, c), lambda i: (i + off, 0))
    blk_l = lambda rr, c: pl.BlockSpec((rr, c), lambda i: (i, 0))
    full = lambda rr, c: pl.BlockSpec((rr, c), lambda i: (0, 0))
    in_specs = [
        blk_g(BE, D), blk_l(BE, D), blk_l(BE, D),
        blk_g(BE, NUM_FES), blk_g(BE, 2),
        full(3 * D, H1), full(1, H1),
        full(H1, D_VAL), full(1, D_VAL),
        full(NUM_FES, H_FC),
        full(H_FC, H_FC * D_VAL), full(D_VAL, H_FC * D_VAL),
        full(H_FC * D_VAL, D),
    ]
    args = [he, hns, hnd, fes, fn, w1, b1, w2, b2, fw1, r, s, w2r]
    if half == 0:
        body = _edge_body0
        aliases = {}
    else:
        body = _edge_body1
        in_specs.append(pl.BlockSpec((8, D), lambda i: (0, 0)))
        args.append(hen_prev)
        aliases = {13: 0}
    return pl.pallas_call(
        body,
        grid=(NBLK_H,),
        in_specs=in_specs,
        out_specs=[pl.BlockSpec((BE, D), lambda i: (i + off, 0)),
                   pl.BlockSpec((BE, D), lambda i: (i, 0))],
        out_shape=[jax.ShapeDtypeStruct((E, D), jnp.float32),
                   jax.ShapeDtypeStruct((EH, D), jnp.float32)],
        input_output_aliases=aliases,
    )(*args)


BN = 1000


def _node_body(hn_r, p0_r, p1_r, w1a_r, w1b_r, b1_r, w2_r, b2_r, hnn_r):
    nt = p0_r[0] + p0_r[1] + p1_r[0] + p1_r[1]
    u = jnp.dot(hn_r[:], w1a_r[:], preferred_element_type=jnp.float32)
    u = u + jnp.dot(nt, w1b_r[:], preferred_element_type=jnp.float32)
    u = jnp.maximum(u + b1_r[:], 0.0)
    hnn_r[:] = hn_r[:] + jnp.dot(u, w2_r[:],
                                 preferred_element_type=jnp.float32) + b2_r[:]


def _node_call(hn, part0, part1, w1a, w1b, b1, w2, b2):
    return pl.pallas_call(
        _node_body,
        grid=(N_NODES // BN,),
        in_specs=[
            pl.BlockSpec((BN, D), lambda i: (i, 0)),
            pl.BlockSpec((NC, BN, D), lambda i: (0, i, 0)),
            pl.BlockSpec((NC, BN, D), lambda i: (0, i, 0)),
            pl.BlockSpec((D, H1), lambda i: (0, 0)),
            pl.BlockSpec((D, H1), lambda i: (0, 0)),
            pl.BlockSpec((1, H1), lambda i: (0, 0)),
            pl.BlockSpec((H1, D), lambda i: (0, 0)),
            pl.BlockSpec((1, D), lambda i: (0, 0)),
        ],
        out_specs=pl.BlockSpec((BN, D), lambda i: (i, 0)),
        out_shape=jax.ShapeDtypeStruct((N_NODES, D), jnp.float32),
    )(hn, part0, part1, w1a, w1b, b1, w2, b2)


@jax.jit
def kernel(hn, he, edge_index, fe, fes, norm,
           ev_W1, ev_b1, ev_W2, ev_b2, fc_W1, fc_W2,
           nu_W1, nu_b1, nu_W2, nu_b2):
    src = edge_index[0]
    dst = edge_index[1]
    fn = jnp.concatenate([fe, norm[:, None]], axis=1)
    bf = jnp.bfloat16
    ew = dict(
        w1=ev_W1.astype(bf), b1=ev_b1.reshape(1, H1),
        w2=ev_W2.astype(bf), b2=ev_b2.reshape(1, D_VAL),
        fw1=fc_W1.astype(bf),
        r=jnp.kron(jnp.eye(H_FC, dtype=jnp.float32),
                   jnp.ones((1, D_VAL), jnp.float32)).astype(bf),
        s=jnp.tile(jnp.eye(D_VAL, dtype=jnp.float32), (1, H_FC)).astype(bf),
        w2r=fc_W2.reshape(H_FC * D_VAL, D).astype(bf))
    zeros = jnp.zeros((NR_CHUNK, D), jnp.float32)

    hns0, hnd0 = _gathers[0](hn, src, dst)
    hns1, hnd1 = _gathers[1](hn, src, dst)
    hen0, henw0 = _edge_call(0, he, hns0, hnd0, fes, fn, **ew)
    part0 = _scatters[0](henw0, dst, zeros)
    hen, henw1 = _edge_call(1, he, hns1, hnd1, fes, fn, **ew, hen_prev=hen0)
    part1 = _scatters[1](henw1, dst, zeros)
    hnn = _node_call(hn, part0, part1,
                     nu_W1[:D], nu_W1[D:], nu_b1.reshape(1, H1),
                     nu_W2, nu_b2.reshape(1, D))
    return hnn, hen


# speedup vs baseline: 1.6824x; 1.0002x over previous
import functools
import jax
import jax.numpy as jnp
from jax import lax
from jax.experimental import pallas as pl
from jax.experimental.pallas import tpu as pltpu
from jax.experimental.pallas import tpu_sc as plsc

N_NODES = 10000
E = 160000
D = 128
D_VAL = 16
NUM_FES = 16
H1 = 512
H_FC = 64
CHUNK = 128
NC, NS = 2, 16
NW = NC * NS
NSPLIT = 2
EH = E // NSPLIT
NCH = EH // CHUNK
ITERS_H = (NCH + NW - 1) // NW
NR_CHUNK = 80
NRCHUNKS = N_NODES // NR_CHUNK
NR_ITERS = (NRCHUNKS + NS - 1) // NS

_mesh = plsc.VectorSubcoreMesh(core_axis_name="c", subcore_axis_name="s")


def _gather_body(chunk_lo, hn_hbm, src_hbm, dst_hbm, hns_hbm, hnd_hbm,
                 idx_s, idx_d, rows_s, rows_d, gsem0, gsem1):
    cid = lax.axis_index("c")
    sid = lax.axis_index("s")
    wid = sid * NC + cid
    gsems = (gsem0, gsem1)

    def chunk(step):
        return jnp.minimum(wid + step * NW, NCH - 1)

    def fire(step, b):
        base = (chunk(step) + chunk_lo) * CHUNK
        pltpu.sync_copy(src_hbm.at[pl.ds(base, CHUNK)], idx_s.at[b])
        pltpu.sync_copy(dst_hbm.at[pl.ds(base, CHUNK)], idx_d.at[b])
        pltpu.async_copy(hn_hbm.at[idx_s.at[b]], rows_s.at[b], gsems[b])
        pltpu.async_copy(hn_hbm.at[idx_d.at[b]], rows_d.at[b], gsems[b])

    def complete(step, b):
        base = chunk(step) * CHUNK
        pltpu.make_async_copy(hn_hbm.at[idx_s.at[b]], rows_s.at[b],
                              gsems[b]).wait()
        pltpu.make_async_copy(hn_hbm.at[idx_d.at[b]], rows_d.at[b],
                              gsems[b]).wait()
        pltpu.sync_copy(rows_s.at[b], hns_hbm.at[pl.ds(base, CHUNK)])
        pltpu.sync_copy(rows_d.at[b], hnd_hbm.at[pl.ds(base, CHUNK)])

    def body(i, carry):
        s0 = 2 * i
        fire(s0, 0)

        @pl.when(i > 0)
        def _():
            complete(s0 - 1, 1)

        fire(s0 + 1, 1)
        complete(s0, 0)
        return carry

    lax.fori_loop(0, ITERS_H // 2, body, 0)
    complete(ITERS_H - 1, 1)


def _make_gather(chunk_lo):
    return pl.kernel(
        functools.partial(_gather_body, chunk_lo),
        mesh=_mesh,
        out_type=[jax.ShapeDtypeStruct((EH, D), jnp.float32),
                  jax.ShapeDtypeStruct((EH, D), jnp.float32)],
        scratch_types=[
            pltpu.VMEM((2, CHUNK), jnp.int32),
            pltpu.VMEM((2, CHUNK), jnp.int32),
            pltpu.VMEM((2, CHUNK, D), jnp.float32),
            pltpu.VMEM((2, CHUNK, D), jnp.float32),
            pltpu.SemaphoreType.DMA,
            pltpu.SemaphoreType.DMA,
        ],
    )


_gathers = [_make_gather(k * NCH) for k in range(NSPLIT)]


def _scatter_body(chunk_lo, henw_hbm, dst_hbm, zeros_hbm, out_hbm, idx2, rows,
                  lsem0, lsem1, acc):
    cid = lax.axis_index("c")
    sid = lax.axis_index("s")
    wid = sid * NC + cid
    lsems = (lsem0, lsem1)

    def zbody(i, carry):
        c = sid + i * NS

        @pl.when(c < NRCHUNKS)
        def _():
            pltpu.sync_copy(zeros_hbm, acc.at[pl.ds(c * NR_CHUNK, NR_CHUNK)])

        return carry

    lax.fori_loop(0, NR_ITERS, zbody, 0)
    plsc.subcore_barrier()

    def fire(step, b):
        c = wid + step * NW

        @pl.when(c < NCH)
        def _():
            pltpu.async_copy(dst_hbm.at[pl.ds((c + chunk_lo) * CHUNK, CHUNK)],
                             idx2.at[b], lsems[b])
            pltpu.async_copy(henw_hbm.at[pl.ds(c * CHUNK, CHUNK)],
                             rows.at[b], lsems[b])

    def complete(step, b):
        c = wid + step * NW

        @pl.when(c < NCH)
        def _():
            pltpu.make_async_copy(dst_hbm.at[pl.ds(0, CHUNK)], idx2.at[b],
                                  lsems[b]).wait()
            pltpu.make_async_copy(henw_hbm.at[pl.ds(0, CHUNK)], rows.at[b],
                                  lsems[b]).wait()
            pltpu.sync_copy(rows.at[b], acc.at[idx2.at[b]], add=True)

    def body(i, carry):
        s0 = 2 * i
        fire(s0, 0)

        @pl.when(i > 0)
        def _():
            complete(s0 - 1, 1)

        fire(s0 + 1, 1)
        complete(s0, 0)
        return carry

    lax.fori_loop(0, ITERS_H // 2, body, 0)
    complete(ITERS_H - 1, 1)
    plsc.subcore_barrier()

    def dbody(i, carry):
        c = sid + i * NS

        @pl.when(c < NRCHUNKS)
        def _():
            pltpu.sync_copy(acc.at[pl.ds(c * NR_CHUNK, NR_CHUNK)],
                            out_hbm.at[cid, pl.ds(c * NR_CHUNK, NR_CHUNK)])

        return carry

    lax.fori_loop(0, NR_ITERS, dbody, 0)


def _make_scatter(chunk_lo):
    return pl.kernel(
        functools.partial(_scatter_body, chunk_lo),
        mesh=_mesh,
        out_type=jax.ShapeDtypeStruct((NC, N_NODES, D), jnp.float32),
        scratch_types=[
            pltpu.VMEM((2, CHUNK), jnp.int32),
            pltpu.VMEM((2, CHUNK, D), jnp.float32),
            pltpu.SemaphoreType.DMA,
            pltpu.SemaphoreType.DMA,
            pltpu.VMEM_SHARED((N_NODES, D), jnp.float32),
        ],
    )


_scatters = [_make_scatter(k * NCH) for k in range(NSPLIT)]


BE = 3200
NBLK_H = EH // BE


def _edge_math(he, hns, hnd, fesT, fe, nm, w1, b1, w2, b2, fw1, r, s, w2r):
    bf = jnp.bfloat16
    z = jnp.concatenate(
        [he.astype(bf), hns.astype(bf), hnd.astype(bf)], axis=1)
    t = jnp.dot(z, w1, preferred_element_type=jnp.float32)
    t = jnp.maximum(t + b1, 0.0).astype(bf)
    v = jnp.dot(t, w2, preferred_element_type=jnp.float32) + b2
    h = jnp.maximum(
        lax.dot_general(fesT.astype(bf), fw1, (((0,), (0,)), ((), ())),
                        preferred_element_type=jnp.float32) * 0.25,
        0.0).astype(bf)
    v_s = (v * (fe * (1.0 / 32.0))).astype(bf)
    h_rep = jnp.dot(h, r, preferred_element_type=jnp.float32)
    v_tile = jnp.dot(v_s, s, preferred_element_type=jnp.float32)
    p = (h_rep * v_tile).astype(bf)
    heu = jnp.dot(p, w2r, preferred_element_type=jnp.float32)
    hen = he + heu
    return hen, hen * nm


def _edge_body0(he_r, hns_r, hnd_r, fesT_r, fe_r, nm_r,
                w1_r, b1_r, w2_r, b2_r, fw1_r, r_r, s_r, w2r_r,
                hen_r, henw_r):
    hen, henw = _edge_math(he_r[:], hns_r[:], hnd_r[:], fesT_r[:], fe_r[:],
                           nm_r[:], w1_r[:], b1_r[:], w2_r[:], b2_r[:],
                           fw1_r[:], r_r[:], s_r[:], w2r_r[:])
    hen_r[:] = hen
    henw_r[:] = henw


def _edge_body1(he_r, hns_r, hnd_r, fesT_r, fe_r, nm_r,
                w1_r, b1_r, w2_r, b2_r, fw1_r, r_r, s_r, w2r_r, prev_r,
                hen_r, henw_r):
    hen, henw = _edge_math(he_r[:], hns_r[:], hnd_r[:], fesT_r[:], fe_r[:],
                           nm_r[:], w1_r[:], b1_r[:], w2_r[:], b2_r[:],
                           fw1_r[:], r_r[:], s_r[:], w2r_r[:])
    hen_r[:] = hen
    henw_r[:] = henw


def _edge_call(half, he, hns, hnd, fesT, fe, nm, w1, b1, w2, b2, fw1, r, s,
               w2r, hen_prev=None):
    off = half * NBLK_H
    blk_g = lambda rr, c: pl.BlockSpec((rr, c), lambda i: (i + off, 0))
    blk_l = lambda rr, c: pl.BlockSpec((rr, c), lambda i: (i, 0))
    full = lambda rr, c: pl.BlockSpec((rr, c), lambda i: (0, 0))
    in_specs = [
        blk_g(BE, D), blk_l(BE, D), blk_l(BE, D),
        pl.BlockSpec((NUM_FES, BE), lambda i: (0, i + off)),
        blk_g(BE, 1), blk_g(BE, 1),
        full(3 * D, H1), full(1, H1),
        full(H1, D_VAL), full(1, D_VAL),
        full(NUM_FES, H_FC),
        full(H_FC, H_FC * D_VAL), full(D_VAL, H_FC * D_VAL),
        full(H_FC * D_VAL, D),
    ]
    args = [he, hns, hnd, fesT, fe, nm, w1, b1, w2, b2, fw1, r, s, w2r]
    if half == 0:
        body = _edge_body0
        aliases = {}
    else:
        body = _edge_body1
        in_specs.append(pl.BlockSpec((8, D), lambda i: (0, 0)))
        args.append(hen_prev)
        aliases = {14: 0}
    return pl.pallas_call(
        body,
        grid=(NBLK_H,),
        in_specs=in_specs,
        out_specs=[pl.BlockSpec((BE, D), lambda i: (i + off, 0)),
                   pl.BlockSpec((BE, D), lambda i: (i, 0))],
        out_shape=[jax.ShapeDtypeStruct((E, D), jnp.float32),
                   jax.ShapeDtypeStruct((EH, D), jnp.float32)],
        input_output_aliases=aliases,
    )(*args)


BN = 1000


def _node_body(hn_r, p0_r, p1_r, w1a_r, w1b_r, b1_r, w2_r, b2_r, hnn_r):
    nt = p0_r[0] + p0_r[1] + p1_r[0] + p1_r[1]
    u = jnp.dot(hn_r[:], w1a_r[:], preferred_element_type=jnp.float32)
    u = u + jnp.dot(nt, w1b_r[:], preferred_element_type=jnp.float32)
    u = jnp.maximum(u + b1_r[:], 0.0)
    hnn_r[:] = hn_r[:] + jnp.dot(u, w2_r[:],
                                 preferred_element_type=jnp.float32) + b2_r[:]


def _node_call(hn, part0, part1, w1a, w1b, b1, w2, b2):
    return pl.pallas_call(
        _node_body,
        grid=(N_NODES // BN,),
        in_specs=[
            pl.BlockSpec((BN, D), lambda i: (i, 0)),
            pl.BlockSpec((NC, BN, D), lambda i: (0, i, 0)),
            pl.BlockSpec((NC, BN, D), lambda i: (0, i, 0)),
            pl.BlockSpec((D, H1), lambda i: (0, 0)),
            pl.BlockSpec((D, H1), lambda i: (0, 0)),
            pl.BlockSpec((1, H1), lambda i: (0, 0)),
            pl.BlockSpec((H1, D), lambda i: (0, 0)),
            pl.BlockSpec((1, D), lambda i: (0, 0)),
        ],
        out_specs=pl.BlockSpec((BN, D), lambda i: (i, 0)),
        out_shape=jax.ShapeDtypeStruct((N_NODES, D), jnp.float32),
    )(hn, part0, part1, w1a, w1b, b1, w2, b2)


@jax.jit
def kernel(hn, he, edge_index, fe, fes, norm,
           ev_W1, ev_b1, ev_W2, ev_b2, fc_W1, fc_W2,
           nu_W1, nu_b1, nu_W2, nu_b2):
    src = edge_index[0]
    dst = edge_index[1]
    fesT = fes.T
    nm = norm.reshape(E, 1)
    bf = jnp.bfloat16
    ew = dict(
        w1=ev_W1.astype(bf), b1=ev_b1.reshape(1, H1),
        w2=ev_W2.astype(bf), b2=ev_b2.reshape(1, D_VAL),
        fw1=fc_W1.astype(bf),
        r=jnp.kron(jnp.eye(H_FC, dtype=jnp.float32),
                   jnp.ones((1, D_VAL), jnp.float32)).astype(bf),
        s=jnp.tile(jnp.eye(D_VAL, dtype=jnp.float32), (1, H_FC)).astype(bf),
        w2r=fc_W2.reshape(H_FC * D_VAL, D).astype(bf))
    zeros = jnp.zeros((NR_CHUNK, D), jnp.float32)

    hns0, hnd0 = _gathers[0](hn, src, dst)
    hns1, hnd1 = _gathers[1](hn, src, dst)
    hen0, henw0 = _edge_call(0, he, hns0, hnd0, fesT, fe, nm, **ew)
    part0 = _scatters[0](henw0, dst, zeros)
    hen, henw1 = _edge_call(1, he, hns1, hnd1, fesT, fe, nm, **ew,
                            hen_prev=hen0)
    part1 = _scatters[1](henw1, dst, zeros)
    hnn = _node_call(hn, part0, part1,
                     nu_W1[:D], nu_W1[D:], nu_b1.reshape(1, H1),
                     nu_W2, nu_b2.reshape(1, D))
    return hnn, hen


# speedup vs baseline: 1.9625x; 1.1665x over previous
import functools
import jax
import jax.numpy as jnp
from jax import lax
from jax.experimental import pallas as pl
from jax.experimental.pallas import tpu as pltpu
from jax.experimental.pallas import tpu_sc as plsc

N_NODES = 10000
E = 160000
D = 128
D_VAL = 16
NUM_FES = 16
H1 = 512
H_FC = 64
CHUNK = 128
NC, NS = 2, 16
NW = NC * NS
NSPLIT = 2
EH = E // NSPLIT
NCH = EH // CHUNK
ITERS_H = (NCH + NW - 1) // NW
NR_CHUNK = 80
NRCHUNKS = N_NODES // NR_CHUNK
NR_ITERS = (NRCHUNKS + NS - 1) // NS

_mesh = plsc.VectorSubcoreMesh(core_axis_name="c", subcore_axis_name="s")


def _gather_body(chunk_lo, hn_hbm, src_hbm, dst_hbm, hns_hbm, hnd_hbm,
                 idx_s, idx_d, rows_s, rows_d, gsem0, gsem1):
    cid = lax.axis_index("c")
    sid = lax.axis_index("s")
    wid = sid * NC + cid
    gsems = (gsem0, gsem1)

    def chunk(step):
        return jnp.minimum(wid + step * NW, NCH - 1)

    def fire(step, b):
        base = (chunk(step) + chunk_lo) * CHUNK
        pltpu.sync_copy(src_hbm.at[pl.ds(base, CHUNK)], idx_s.at[b])
        pltpu.sync_copy(dst_hbm.at[pl.ds(base, CHUNK)], idx_d.at[b])
        pltpu.async_copy(hn_hbm.at[idx_s.at[b]], rows_s.at[b], gsems[b])
        pltpu.async_copy(hn_hbm.at[idx_d.at[b]], rows_d.at[b], gsems[b])

    def complete(step, b):
        base = chunk(step) * CHUNK
        pltpu.make_async_copy(hn_hbm.at[idx_s.at[b]], rows_s.at[b],
                              gsems[b]).wait()
        pltpu.make_async_copy(hn_hbm.at[idx_d.at[b]], rows_d.at[b],
                              gsems[b]).wait()
        pltpu.sync_copy(rows_s.at[b], hns_hbm.at[pl.ds(base, CHUNK)])
        pltpu.sync_copy(rows_d.at[b], hnd_hbm.at[pl.ds(base, CHUNK)])

    def body(i, carry):
        s0 = 2 * i
        fire(s0, 0)

        @pl.when(i > 0)
        def _():
            complete(s0 - 1, 1)

        fire(s0 + 1, 1)
        complete(s0, 0)
        return carry

    lax.fori_loop(0, ITERS_H // 2, body, 0)
    complete(ITERS_H - 1, 1)


def _make_gather(chunk_lo):
    return pl.kernel(
        functools.partial(_gather_body, chunk_lo),
        mesh=_mesh,
        out_type=[jax.ShapeDtypeStruct((EH, D), jnp.float32),
                  jax.ShapeDtypeStruct((EH, D), jnp.float32)],
        scratch_types=[
            pltpu.VMEM((2, CHUNK), jnp.int32),
            pltpu.VMEM((2, CHUNK), jnp.int32),
            pltpu.VMEM((2, CHUNK, D), jnp.float32),
            pltpu.VMEM((2, CHUNK, D), jnp.float32),
            pltpu.SemaphoreType.DMA,
            pltpu.SemaphoreType.DMA,
        ],
    )


_gathers = [_make_gather(k * NCH) for k in range(NSPLIT)]


def _scatter_body(chunk_lo, henw_hbm, dst_hbm, zeros_hbm, out_hbm, idx2, rows,
                  lsem0, lsem1, acc):
    cid = lax.axis_index("c")
    sid = lax.axis_index("s")
    wid = sid * NC + cid
    lsems = (lsem0, lsem1)

    def zbody(i, carry):
        c = sid + i * NS

        @pl.when(c < NRCHUNKS)
        def _():
            pltpu.sync_copy(zeros_hbm, acc.at[pl.ds(c * NR_CHUNK, NR_CHUNK)])

        return carry

    lax.fori_loop(0, NR_ITERS, zbody, 0)
    plsc.subcore_barrier()

    def fire(step, b):
        c = wid + step * NW

        @pl.when(c < NCH)
        def _():
            pltpu.async_copy(dst_hbm.at[pl.ds((c + chunk_lo) * CHUNK, CHUNK)],
                             idx2.at[b], lsems[b])
            pltpu.async_copy(henw_hbm.at[pl.ds(c * CHUNK, CHUNK)],
                             rows.at[b], lsems[b])

    def complete(step, b):
        c = wid + step * NW

        @pl.when(c < NCH)
        def _():
            pltpu.make_async_copy(dst_hbm.at[pl.ds(0, CHUNK)], idx2.at[b],
                                  lsems[b]).wait()
            pltpu.make_async_copy(henw_hbm.at[pl.ds(0, CHUNK)], rows.at[b],
                                  lsems[b]).wait()
            pltpu.sync_copy(rows.at[b], acc.at[idx2.at[b]], add=True)

    def body(i, carry):
        s0 = 2 * i
        fire(s0, 0)

        @pl.when(i > 0)
        def _():
            complete(s0 - 1, 1)

        fire(s0 + 1, 1)
        complete(s0, 0)
        return carry

    lax.fori_loop(0, ITERS_H // 2, body, 0)
    complete(ITERS_H - 1, 1)
    plsc.subcore_barrier()

    def dbody(i, carry):
        c = sid + i * NS

        @pl.when(c < NRCHUNKS)
        def _():
            pltpu.sync_copy(acc.at[pl.ds(c * NR_CHUNK, NR_CHUNK)],
                            out_hbm.at[cid, pl.ds(c * NR_CHUNK, NR_CHUNK)])

        return carry

    lax.fori_loop(0, NR_ITERS, dbody, 0)


def _make_scatter(chunk_lo):
    return pl.kernel(
        functools.partial(_scatter_body, chunk_lo),
        mesh=_mesh,
        out_type=jax.ShapeDtypeStruct((NC, N_NODES, D), jnp.float32),
        scratch_types=[
            pltpu.VMEM((2, CHUNK), jnp.int32),
            pltpu.VMEM((2, CHUNK, D), jnp.float32),
            pltpu.SemaphoreType.DMA,
            pltpu.SemaphoreType.DMA,
            pltpu.VMEM_SHARED((N_NODES, D), jnp.float32),
        ],
    )


_scatters = [_make_scatter(k * NCH) for k in range(NSPLIT)]


BE = 3200
NBLK_H = EH // BE


def _col(tr):
    return jnp.concatenate([tr[:, g:g + 1] for g in range(tr.shape[1])],
                           axis=0)


def _edge_math(he, hns, hnd, fesT, fe, nm, w1, b1, w2, b2, fw1, r, s, w2r):
    bf = jnp.bfloat16
    z = jnp.concatenate(
        [he.astype(bf), hns.astype(bf), hnd.astype(bf)], axis=1)
    t = jnp.dot(z, w1, preferred_element_type=jnp.float32)
    t = jnp.maximum(t + b1, 0.0).astype(bf)
    v = jnp.dot(t, w2, preferred_element_type=jnp.float32) + b2
    h = jnp.maximum(
        lax.dot_general(fesT.astype(bf), fw1, (((0,), (0,)), ((), ())),
                        preferred_element_type=jnp.float32) * 0.25,
        0.0).astype(bf)
    v_s = (v * (_col(fe[0]) * (1.0 / 32.0))).astype(bf)
    h_rep = jnp.dot(h, r, preferred_element_type=jnp.float32)
    v_tile = jnp.dot(v_s, s, preferred_element_type=jnp.float32)
    p = (h_rep * v_tile).astype(bf)
    heu = jnp.dot(p, w2r, preferred_element_type=jnp.float32)
    hen = he + heu
    return hen, hen * _col(nm[0])


def _edge_body0(he_r, hns_r, hnd_r, fesT_r, fe_r, nm_r,
                w1_r, b1_r, w2_r, b2_r, fw1_r, r_r, s_r, w2r_r,
                hen_r, henw_r):
    hen, henw = _edge_math(he_r[:], hns_r[:], hnd_r[:], fesT_r[:], fe_r[:],
                           nm_r[:], w1_r[:], b1_r[:], w2_r[:], b2_r[:],
                           fw1_r[:], r_r[:], s_r[:], w2r_r[:])
    hen_r[:] = hen
    henw_r[:] = henw


def _edge_body1(he_r, hns_r, hnd_r, fesT_r, fe_r, nm_r,
                w1_r, b1_r, w2_r, b2_r, fw1_r, r_r, s_r, w2r_r, prev_r,
                hen_r, henw_r):
    hen, henw = _edge_math(he_r[:], hns_r[:], hnd_r[:], fesT_r[:], fe_r[:],
                           nm_r[:], w1_r[:], b1_r[:], w2_r[:], b2_r[:],
                           fw1_r[:], r_r[:], s_r[:], w2r_r[:])
    hen_r[:] = hen
    henw_r[:] = henw


def _edge_call(half, he, hns, hnd, fesT, fe, nm, w1, b1, w2, b2, fw1, r, s,
               w2r, hen_prev=None):
    off = half * NBLK_H
    blk_g = lambda rr, c: pl.BlockSpec((rr, c), lambda i: (i + off, 0))
    blk_l = lambda rr, c: pl.BlockSpec((rr, c), lambda i: (i, 0))
    full = lambda rr, c: pl.BlockSpec((rr, c), lambda i: (0, 0))
    in_specs = [
        blk_g(BE, D), blk_l(BE, D), blk_l(BE, D),
        pl.BlockSpec((NUM_FES, BE), lambda i: (0, i + off)),
        pl.BlockSpec((1, D, BE // D), lambda i: (i + off, 0, 0)),
        pl.BlockSpec((1, D, BE // D), lambda i: (i + off, 0, 0)),
        full(3 * D, H1), full(1, H1),
        full(H1, D_VAL), full(1, D_VAL),
        full(NUM_FES, H_FC),
        full(H_FC, H_FC * D_VAL), full(D_VAL, H_FC * D_VAL),
        full(H_FC * D_VAL, D),
    ]
    args = [he, hns, hnd, fesT, fe, nm, w1, b1, w2, b2, fw1, r, s, w2r]
    if half == 0:
        body = _edge_body0
        aliases = {}
    else:
        body = _edge_body1
        in_specs.append(pl.BlockSpec((8, D), lambda i: (0, 0)))
        args.append(hen_prev)
        aliases = {14: 0}
    return pl.pallas_call(
        body,
        grid=(NBLK_H,),
        in_specs=in_specs,
        out_specs=[pl.BlockSpec((BE, D), lambda i: (i + off, 0)),
                   pl.BlockSpec((BE, D), lambda i: (i, 0))],
        out_shape=[jax.ShapeDtypeStruct((E, D), jnp.float32),
                   jax.ShapeDtypeStruct((EH, D), jnp.float32)],
        input_output_aliases=aliases,
    )(*args)


BN = 1000


def _node_body(hn_r, p0_r, p1_r, w1a_r, w1b_r, b1_r, w2_r, b2_r, hnn_r):
    nt = p0_r[0] + p0_r[1] + p1_r[0] + p1_r[1]
    u = jnp.dot(hn_r[:], w1a_r[:], preferred_element_type=jnp.float32)
    u = u + jnp.dot(nt, w1b_r[:], preferred_element_type=jnp.float32)
    u = jnp.maximum(u + b1_r[:], 0.0)
    hnn_r[:] = hn_r[:] + jnp.dot(u, w2_r[:],
                                 preferred_element_type=jnp.float32) + b2_r[:]


def _node_call(hn, part0, part1, w1a, w1b, b1, w2, b2):
    return pl.pallas_call(
        _node_body,
        grid=(N_NODES // BN,),
        in_specs=[
            pl.BlockSpec((BN, D), lambda i: (i, 0)),
            pl.BlockSpec((NC, BN, D), lambda i: (0, i, 0)),
            pl.BlockSpec((NC, BN, D), lambda i: (0, i, 0)),
            pl.BlockSpec((D, H1), lambda i: (0, 0)),
            pl.BlockSpec((D, H1), lambda i: (0, 0)),
            pl.BlockSpec((1, H1), lambda i: (0, 0)),
            pl.BlockSpec((H1, D), lambda i: (0, 0)),
            pl.BlockSpec((1, D), lambda i: (0, 0)),
        ],
        out_specs=pl.BlockSpec((BN, D), lambda i: (i, 0)),
        out_shape=jax.ShapeDtypeStruct((N_NODES, D), jnp.float32),
    )(hn, part0, part1, w1a, w1b, b1, w2, b2)


@jax.jit
def kernel(hn, he, edge_index, fe, fes, norm,
           ev_W1, ev_b1, ev_W2, ev_b2, fc_W1, fc_W2,
           nu_W1, nu_b1, nu_W2, nu_b2):
    src = edge_index[0]
    dst = edge_index[1]
    fesT = fes.T
    fe2 = fe.reshape(E // BE, BE // D, D).transpose(0, 2, 1)
    nm2 = norm.reshape(E // BE, BE // D, D).transpose(0, 2, 1)
    bf = jnp.bfloat16
    ew = dict(
        w1=ev_W1.astype(bf), b1=ev_b1.reshape(1, H1),
        w2=ev_W2.astype(bf), b2=ev_b2.reshape(1, D_VAL),
        fw1=fc_W1.astype(bf),
        r=jnp.kron(jnp.eye(H_FC, dtype=jnp.float32),
                   jnp.ones((1, D_VAL), jnp.float32)).astype(bf),
        s=jnp.tile(jnp.eye(D_VAL, dtype=jnp.float32), (1, H_FC)).astype(bf),
        w2r=fc_W2.reshape(H_FC * D_VAL, D).astype(bf))
    zeros = jnp.zeros((NR_CHUNK, D), jnp.float32)

    hns0, hnd0 = _gathers[0](hn, src, dst)
    hns1, hnd1 = _gathers[1](hn, src, dst)
    hen0, henw0 = _edge_call(0, he, hns0, hnd0, fesT, fe2, nm2, **ew)
    part0 = _scatters[0](henw0, dst, zeros)
    hen, henw1 = _edge_call(1, he, hns1, hnd1, fesT, fe2, nm2, **ew,
                            hen_prev=hen0)
    part1 = _scatters[1](henw1, dst, zeros)
    hnn = _node_call(hn, part0, part1,
                     nu_W1[:D], nu_W1[D:], nu_b1.reshape(1, H1),
                     nu_W2, nu_b2.reshape(1, D))
    return hnn, hen
